# baseline (speedup 1.0000x reference)
import functools
import jax
import jax.numpy as jnp
from jax import lax
from jax.experimental import pallas as pl
from jax.experimental.pallas import tpu as pltpu

N_NODES = 5120
N_EDGES = 15360
N_GRAPHS = 64
F = 1262
PF = 1280


def _pad2(a, r, c):
    return jnp.zeros((r, c), a.dtype).at[: a.shape[0], : a.shape[1]].set(a)


def _padb(b, c):
    z = jnp.zeros((8, c), b.dtype)
    return z.at[0, : b.shape[0]].set(b)


def _ctab_body(emb_ref, wenc_ref, benc_ref, wc_ref, out_ref):
    t = jnp.dot(emb_ref[...], wenc_ref[...], preferred_element_type=jnp.float32)
    t = t + benc_ref[0:1, :]
    out_ref[...] = jnp.dot(t, wc_ref[...], preferred_element_type=jnp.float32)


def _ctab(emb, wenc, benc, wc):
    return pl.pallas_call(
        _ctab_body,
        out_shape=jax.ShapeDtypeStruct((32, PF), jnp.float32),
    )(emb, wenc, benc, wc)


def _node_pre_body(x_ref, w1_ref, b1_ref, wd_ref, bd_ref, ws_ref, h0_ref, hd_ref, hs_ref):
    h0 = jnp.dot(x_ref[...], w1_ref[...], preferred_element_type=jnp.float32)
    h0 = jnp.maximum(h0 + b1_ref[0:1, :], 0.0)
    h0_ref[...] = h0
    hb = h0.astype(jnp.bfloat16)
    hd_ref[...] = jnp.dot(hb, wd_ref[...], preferred_element_type=jnp.float32) + bd_ref[0:1, :]
    hs_ref[...] = jnp.dot(hb, ws_ref[...], preferred_element_type=jnp.float32)


def _node_pre(x, w1, b1, wd, bd, ws):
    bm = 512
    grid = (N_NODES // bm,)
    blk = pl.BlockSpec((bm, PF), lambda i: (i, 0))
    wspec = pl.BlockSpec((PF, PF), lambda i: (0, 0))
    bspec = pl.BlockSpec((8, PF), lambda i: (0, 0))
    return pl.pallas_call(
        _node_pre_body,
        grid=grid,
        in_specs=[blk, wspec, bspec, wspec, bspec, wspec],
        out_specs=[blk, blk, blk],
        out_shape=[jax.ShapeDtypeStruct((N_NODES, PF), jnp.float32)] * 3,
    )(x, w1, b1, wd, bd, ws)


def _pre_chain_body(m0_ref, attr_ref, ctab_ref, w1, b1, w2, b2, w3, b3, w4, b4, out_ref):
    a = attr_ref[0, 0, :]
    oh = (a[:, None] == lax.broadcasted_iota(jnp.int32, (a.shape[0], 32), 1)).astype(jnp.bfloat16)
    m = m0_ref[...] + jnp.dot(oh, ctab_ref[...], preferred_element_type=jnp.float32)
    for w_ref, b_ref in ((w1, b1), (w2, b2), (w3, b3), (w4, b4)):
        m = jnp.maximum(m, 0.0).astype(jnp.bfloat16)
        m = jnp.dot(m, w_ref[...], preferred_element_type=jnp.float32) + b_ref[0:1, :]
    out_ref[...] = m


def _pre_chain(m0, attr3, ctab, ws, bs):
    bm = 768
    grid = (N_EDGES // bm,)
    blk = pl.BlockSpec((bm, PF), lambda i: (i, 0))
    aspec = pl.BlockSpec((1, 1, bm), lambda i: (i, 0, 0))
    cspec = pl.BlockSpec((32, PF), lambda i: (0, 0))
    wspec = pl.BlockSpec((PF, PF), lambda i: (0, 0))
    bspec = pl.BlockSpec((8, PF), lambda i: (0, 0))
    in_specs = [blk, aspec, cspec]
    args = [m0, attr3, ctab]
    for w, b in zip(ws, bs):
        in_specs += [wspec, bspec]
        args += [w, b]
    return pl.pallas_call(
        _pre_chain_body,
        grid=grid,
        in_specs=in_specs,
        out_specs=blk,
        out_shape=jax.ShapeDtypeStruct((N_EDGES, PF), jnp.float32),
    )(*args)


def _post0_body(h0_ref, s_ref, sq_ref, mn_ref, mx_ref, cnt_ref, aw_ref,
                wh, wsum, wmean, wmin, wmax, wstd, b_ref, out_ref):
    cnt = cnt_ref[...][:, 0:1]
    cnt_safe = jnp.maximum(cnt, 1.0)
    has = (cnt > 0.0).astype(jnp.float32)
    s = s_ref[...]
    mean = s / cnt_safe
    mn = mn_ref[...] * has
    mx = mx_ref[...] * has
    msq = sq_ref[...] / cnt_safe
    var = jnp.maximum(msq - mean * mean, 0.0)
    std = jnp.sqrt(var + 1e-5)
    a0 = aw_ref[0, 0]
    a1 = aw_ref[0, 1]
    a2 = aw_ref[0, 2]
    a3 = aw_ref[0, 3]
    a4 = aw_ref[0, 4]
    bf = jnp.bfloat16
    acc = jnp.dot(h0_ref[...].astype(bf), wh[...], preferred_element_type=jnp.float32)
    acc += jnp.dot((a0 * s).astype(bf), wsum[...], preferred_element_type=jnp.float32)
    acc += jnp.dot((a1 * mean).astype(bf), wmean[...], preferred_element_type=jnp.float32)
    acc += jnp.dot((a2 * mn).astype(bf), wmin[...], preferred_element_type=jnp.float32)
    acc += jnp.dot((a3 * mx).astype(bf), wmax[...], preferred_element_type=jnp.float32)
    acc += jnp.dot((a4 * std).astype(bf), wstd[...], preferred_element_type=jnp.float32)
    out_ref[...] = acc + b_ref[0:1, :]


def _post0(h0, s, sq, mn, mx, cnt128, aw, wslices, b0):
    bm = 256
    grid = (N_NODES // bm,)
    blk = pl.BlockSpec((bm, PF), lambda i: (i, 0))
    cspec = pl.BlockSpec((bm, 128), lambda i: (i, 0))
    awspec = pl.BlockSpec((8, 128), lambda i: (0, 0))
    wspec = pl.BlockSpec((PF, PF), lambda i: (0, 0))
    bspec = pl.BlockSpec((8, PF), lambda i: (0, 0))
    return pl.pallas_call(
        _post0_body,
        grid=grid,
        in_specs=[blk, blk, blk, blk, blk, cspec, awspec] + [wspec] * 6 + [bspec],
        out_specs=blk,
        out_shape=jax.ShapeDtypeStruct((N_NODES, PF), jnp.float32),
    )(h0, s, sq, mn, mx, cnt128, aw, *wslices, b0)


def _post_chain_body(x_ref, w1, b1, w2, b2, w3, b3, w4, b4, wl, bl, out_ref, part_ref):
    m = x_ref[...]
    for w_ref, b_ref in ((w1, b1), (w2, b2), (w3, b3), (w4, b4)):
        m = jnp.maximum(m, 0.0).astype(jnp.bfloat16)
        m = jnp.dot(m, w_ref[...], preferred_element_type=jnp.float32) + b_ref[0:1, :]
    m = jnp.dot(m.astype(jnp.bfloat16), wl[...], preferred_element_type=jnp.float32) + bl[0:1, :]
    out_ref[...] = m
    part_ref[0, 0, :] = jnp.sum(m, axis=0)
    part_ref[0, 1, :] = jnp.sum(m * m, axis=0)


def _post_chain(x, ws, bs, wl, bl):
    bm = 512
    nb = N_NODES // bm
    grid = (nb,)
    blk = pl.BlockSpec((bm, PF), lambda i: (i, 0))
    wspec = pl.BlockSpec((PF, PF), lambda i: (0, 0))
    bspec = pl.BlockSpec((8, PF), lambda i: (0, 0))
    in_specs = [blk]
    args = [x]
    for w, b in zip(ws, bs):
        in_specs += [wspec, bspec]
        args += [w, b]
    in_specs += [wspec, bspec]
    args += [wl, bl]
    return pl.pallas_call(
        _post_chain_body,
        grid=grid,
        in_specs=in_specs,
        out_specs=[blk, pl.BlockSpec((1, 2, PF), lambda i: (i, 0, 0))],
        out_shape=[jax.ShapeDtypeStruct((N_NODES, PF), jnp.float32),
                   jax.ShapeDtypeStruct((nb, 2, PF), jnp.float32)],
    )(*args)


def _finale_body(x_ref, part_ref, batch_ref, g_ref, be_ref, w1, b1, w2, b2, w3, b3,
                 xf_ref, pool_ref):
    i = pl.program_id(0)
    colsum = jnp.sum(part_ref[:, 0, :], axis=0, keepdims=True)
    colsq = jnp.sum(part_ref[:, 1, :], axis=0, keepdims=True)
    mu = colsum / float(N_NODES)
    var = colsq / float(N_NODES) - mu * mu
    rstd = lax.rsqrt(var + 1e-5)
    h = (x_ref[...] - mu) * rstd * g_ref[0:1, :] + be_ref[0:1, :]
    h = jnp.maximum(h, 0.0)
    b = batch_ref[0, 0, :]
    b2d = jnp.broadcast_to(b[None, :], (N_GRAPHS, b.shape[0]))
    g2d = lax.broadcasted_iota(jnp.int32, (N_GRAPHS, b.shape[0]), 0)
    oh = (b2d == g2d).astype(jnp.float32)

    @pl.when(i == 0)
    def _():
        pool_ref[...] = jnp.zeros_like(pool_ref)

    pool_ref[...] += jnp.dot(oh, h, preferred_element_type=jnp.float32)
    f = jnp.maximum(jnp.dot(h.astype(jnp.bfloat16), w1[...], preferred_element_type=jnp.float32) + b1[0:1, :], 0.0)
    f = jnp.maximum(jnp.dot(f.astype(jnp.bfloat16), w2[...], preferred_element_type=jnp.float32) + b2[0:1, :], 0.0)
    xf_ref[...] = jnp.dot(f.astype(jnp.bfloat16), w3[...], preferred_element_type=jnp.float32) + b3[0:1, :]


def _finale(x, part, batch3, gamma, beta, w1, b1, w2, b2, w3, b3):
    bm = 512
    nb = N_NODES // bm
    grid = (nb,)
    blk = pl.BlockSpec((bm, PF), lambda i: (i, 0))
    return pl.pallas_call(
        _finale_body,
        grid=grid,
        in_specs=[blk,
                  pl.BlockSpec((nb, 2, PF), lambda i: (0, 0, 0)),
                  pl.BlockSpec((1, 1, bm), lambda i: (i, 0, 0)),
                  pl.BlockSpec((8, PF), lambda i: (0, 0)),
                  pl.BlockSpec((8, PF), lambda i: (0, 0)),
                  pl.BlockSpec((PF, 640), lambda i: (0, 0)),
                  pl.BlockSpec((8, 640), lambda i: (0, 0)),
                  pl.BlockSpec((640, 128), lambda i: (0, 0)),
                  pl.BlockSpec((8, 128), lambda i: (0, 0)),
                  pl.BlockSpec((128, 128), lambda i: (0, 0)),
                  pl.BlockSpec((8, 128), lambda i: (0, 0))],
        out_specs=[pl.BlockSpec((bm, 128), lambda i: (i, 0)),
                   pl.BlockSpec((N_GRAPHS, PF), lambda i: (0, 0))],
        out_shape=[jax.ShapeDtypeStruct((N_NODES, 128), jnp.float32),
                   jax.ShapeDtypeStruct((N_GRAPHS, PF), jnp.float32)],
    )(x, part, batch3, gamma, beta, w1, b1, w2, b2, w3, b3)


def _mlp2_body(p_ref, w1, b1, w2, b2, w3, b3, out_ref):
    t = jnp.maximum(jnp.dot(p_ref[...], w1[...], preferred_element_type=jnp.float32) + b1[0:1, :], 0.0)
    t = jnp.maximum(jnp.dot(t, w2[...], preferred_element_type=jnp.float32) + b2[0:1, :], 0.0)
    out_ref[...] = jnp.dot(t, w3[...], preferred_element_type=jnp.float32) + b3[0:1, :]


def _mlp2(pool, w1, b1, w2, b2, w3, b3):
    return pl.pallas_call(
        _mlp2_body,
        out_shape=jax.ShapeDtypeStruct((N_GRAPHS, 128), jnp.float32),
    )(pool, w1, b1, w2, b2, w3, b3)


def _edge_gather(hd, hs, dst, src):
    return hd[dst] + hs[src]


def _aggregate(m4, dst):
    s = jax.ops.segment_sum(m4, dst, num_segments=N_NODES)
    sq = jax.ops.segment_sum(m4 * m4, dst, num_segments=N_NODES)
    mn = jax.ops.segment_min(m4, dst, num_segments=N_NODES)
    mx = jax.ops.segment_max(m4, dst, num_segments=N_NODES)
    cnt = jax.ops.segment_sum(jnp.ones((N_EDGES,), jnp.float32), dst, num_segments=N_NODES)
    return s, sq, mn, mx, cnt


def kernel(x, edge_index, edge_attr, batch, params):
    f32 = jnp.float32
    xp = _pad2(x, N_NODES, PF).astype(jnp.bfloat16)
    p = params
    w1 = _pad2(p["mlp1"]["w"], PF, PF)
    b1 = _padb(p["mlp1"]["b"], PF)
    pre0w = p["pre"][0]["w"]
    wd = _pad2(pre0w[:F], PF, PF)
    bd = _padb(p["pre"][0]["b"], PF)
    ws_ = _pad2(pre0w[F:2 * F], PF, PF)
    emb = _pad2(p["edge_emb"], 32, 128)
    wenc = _pad2(p["edge_enc"]["w"], 128, PF)
    benc = _padb(p["edge_enc"]["b"], PF)
    wc = _pad2(pre0w[2 * F:], PF, PF)
    prew = [_pad2(p["pre"][i]["w"], PF, PF) for i in range(1, 5)]
    preb = [_padb(p["pre"][i]["b"], PF) for i in range(1, 5)]
    post0w = p["post"][0]["w"]
    wslices = [_pad2(post0w[i * F:(i + 1) * F], PF, PF) for i in range(6)]
    b0 = _padb(p["post"][0]["b"], PF)
    postw = [_pad2(p["post"][i]["w"], PF, PF) for i in range(1, 5)]
    postb = [_padb(p["post"][i]["b"], PF) for i in range(1, 5)]
    wl = _pad2(p["lin"]["w"], PF, PF)
    bl = _padb(p["lin"]["b"], PF)
    gamma = _padb(p["bn_gamma"], PF)
    beta = _padb(p["bn_beta"], PF)
    m2w1 = _pad2(p["mlp2"][0]["w"], PF, 640)
    m2b1 = _padb(p["mlp2"][0]["b"], 640)
    m2w2 = _pad2(p["mlp2"][1]["w"], 640, 128)
    m2b2 = _padb(p["mlp2"][1]["b"], 128)
    m2w3 = _pad2(p["mlp2"][2]["w"], 128, 128)
    m2b3 = _padb(p["mlp2"][2]["b"], 128)
    m3w1 = _pad2(p["mlp3"][0]["w"], PF, 640)
    m3b1 = _padb(p["mlp3"][0]["b"], 640)
    m3w2 = _pad2(p["mlp3"][1]["w"], 640, 128)
    m3b2 = _padb(p["mlp3"][1]["b"], 128)
    m3w3 = _pad2(p["mlp3"][2]["w"], 128, 128)
    m3b3 = _padb(p["mlp3"][2]["b"], 128)
    aw5 = jax.nn.softmax(p["agg_w"])
    aw = jnp.zeros((8, 128), f32).at[0, :5].set(aw5)

    dst = edge_index[1]
    src = edge_index[0]
    attr3 = edge_attr.astype(jnp.int32).reshape(N_EDGES // 768, 1, 768)
    batch3 = batch.astype(jnp.int32).reshape(N_NODES // 512, 1, 512)

    bf = jnp.bfloat16
    w1 = w1.astype(bf)
    wd = wd.astype(bf)
    ws_ = ws_.astype(bf)
    prew = [w.astype(bf) for w in prew]
    wslices = [w.astype(bf) for w in wslices]
    postw = [w.astype(bf) for w in postw]
    wl = wl.astype(bf)
    m3w1 = m3w1.astype(bf)
    m3w2 = m3w2.astype(bf)
    m3w3 = m3w3.astype(bf)
    ctab = _ctab(emb, wenc, benc, wc).astype(bf)
    h0, hd, hs = _node_pre(xp, w1, b1, wd, bd, ws_)
    m0 = _edge_gather(hd, hs, dst, src)
    m4 = _pre_chain(m0, attr3, ctab, prew, preb)
    s, sq, mn, mx, cnt = _aggregate(m4, dst)
    cnt128 = jnp.broadcast_to(cnt[:, None], (N_NODES, 128))
    o0 = _post0(h0, s, sq, mn, mx, cnt128, aw, wslices, b0)
    out, part = _post_chain(o0, postw, postb, wl, bl)
    xf_pad, pool = _finale(out, part, batch3, gamma, beta, m3w1, m3b1, m3w2, m3b2, m3w3, m3b3)
    xe_pad = _mlp2(pool, m2w1, m2b1, m2w2, m2b2, m2w3, m2b3)
    return xf_pad[:, :3], xe_pad[:, :1]


# speedup vs baseline: 1.1950x; 1.1950x over previous
import functools
import jax
import jax.numpy as jnp
from jax import lax
from jax.experimental import pallas as pl
from jax.experimental.pallas import tpu as pltpu
from jax.experimental.pallas import tpu_sc as plsc

N_NODES = 5120
N_EDGES = 15360
N_GRAPHS = 64
F = 1262
PF = 1280


def _pad2(a, r, c):
    return jnp.zeros((r, c), a.dtype).at[: a.shape[0], : a.shape[1]].set(a)


def _padb(b, c):
    z = jnp.zeros((8, c), b.dtype)
    return z.at[0, : b.shape[0]].set(b)


def _ctab_body(emb_ref, wenc_ref, benc_ref, wc_ref, out_ref):
    t = jnp.dot(emb_ref[...], wenc_ref[...], preferred_element_type=jnp.float32)
    t = t + benc_ref[0:1, :]
    out_ref[...] = jnp.dot(t, wc_ref[...], preferred_element_type=jnp.float32)


def _ctab(emb, wenc, benc, wc):
    return pl.pallas_call(
        _ctab_body,
        out_shape=jax.ShapeDtypeStruct((32, PF), jnp.float32),
    )(emb, wenc, benc, wc)


def _node_pre_body(x_ref, w1_ref, b1_ref, wd_ref, bd_ref, ws_ref, h0_ref, hd_ref, hs_ref):
    h0 = jnp.dot(x_ref[...], w1_ref[...], preferred_element_type=jnp.float32)
    h0 = jnp.maximum(h0 + b1_ref[0:1, :], 0.0)
    h0_ref[...] = h0
    hb = h0.astype(jnp.bfloat16)
    hd_ref[...] = jnp.dot(hb, wd_ref[...], preferred_element_type=jnp.float32) + bd_ref[0:1, :]
    hs_ref[...] = jnp.dot(hb, ws_ref[...], preferred_element_type=jnp.float32)


def _node_pre(x, w1, b1, wd, bd, ws):
    bm = 512
    grid = (N_NODES // bm,)
    blk = pl.BlockSpec((bm, PF), lambda i: (i, 0))
    wspec = pl.BlockSpec((PF, PF), lambda i: (0, 0))
    bspec = pl.BlockSpec((8, PF), lambda i: (0, 0))
    return pl.pallas_call(
        _node_pre_body,
        grid=grid,
        in_specs=[blk, wspec, bspec, wspec, bspec, wspec],
        out_specs=[blk, blk, blk],
        out_shape=[jax.ShapeDtypeStruct((N_NODES, PF), jnp.float32)] * 3,
    )(x, w1, b1, wd, bd, ws)


def _pre_chain_body(m0_ref, attr_ref, ctab_ref, w1, b1, w2, b2, w3, b3, w4, b4, out_ref):
    a = attr_ref[0, 0, :]
    oh = (a[:, None] == lax.broadcasted_iota(jnp.int32, (a.shape[0], 32), 1)).astype(jnp.bfloat16)
    m = m0_ref[...] + jnp.dot(oh, ctab_ref[...], preferred_element_type=jnp.float32)
    for w_ref, b_ref in ((w1, b1), (w2, b2), (w3, b3), (w4, b4)):
        m = jnp.maximum(m, 0.0).astype(jnp.bfloat16)
        m = jnp.dot(m, w_ref[...], preferred_element_type=jnp.float32) + b_ref[0:1, :]
    out_ref[...] = m


def _pre_chain(m0, attr3, ctab, ws, bs):
    bm = 768
    grid = (N_EDGES // bm,)
    blk = pl.BlockSpec((bm, PF), lambda i: (i, 0))
    aspec = pl.BlockSpec((1, 1, bm), lambda i: (i, 0, 0))
    cspec = pl.BlockSpec((32, PF), lambda i: (0, 0))
    wspec = pl.BlockSpec((PF, PF), lambda i: (0, 0))
    bspec = pl.BlockSpec((8, PF), lambda i: (0, 0))
    in_specs = [blk, aspec, cspec]
    args = [m0, attr3, ctab]
    for w, b in zip(ws, bs):
        in_specs += [wspec, bspec]
        args += [w, b]
    return pl.pallas_call(
        _pre_chain_body,
        grid=grid,
        in_specs=in_specs,
        out_specs=blk,
        out_shape=jax.ShapeDtypeStruct((N_EDGES, PF), jnp.float32),
    )(*args)


def _post0_body(h0_ref, s_ref, sq_ref, mn_ref, mx_ref, cnt_ref, aw_ref,
                wh, wsum, wmean, wmin, wmax, wstd, b_ref, out_ref):
    cnt = cnt_ref[...][:, 0:1]
    cnt_safe = jnp.maximum(cnt, 1.0)
    has = (cnt > 0.0).astype(jnp.float32)
    s = s_ref[...]
    mean = s / cnt_safe
    mn = mn_ref[...] * has
    mx = mx_ref[...] * has
    msq = sq_ref[...] / cnt_safe
    var = jnp.maximum(msq - mean * mean, 0.0)
    std = jnp.sqrt(var + 1e-5)
    a0 = aw_ref[0, 0]
    a1 = aw_ref[0, 1]
    a2 = aw_ref[0, 2]
    a3 = aw_ref[0, 3]
    a4 = aw_ref[0, 4]
    bf = jnp.bfloat16
    acc = jnp.dot(h0_ref[...].astype(bf), wh[...], preferred_element_type=jnp.float32)
    acc += jnp.dot((a0 * s).astype(bf), wsum[...], preferred_element_type=jnp.float32)
    acc += jnp.dot((a1 * mean).astype(bf), wmean[...], preferred_element_type=jnp.float32)
    acc += jnp.dot((a2 * mn).astype(bf), wmin[...], preferred_element_type=jnp.float32)
    acc += jnp.dot((a3 * mx).astype(bf), wmax[...], preferred_element_type=jnp.float32)
    acc += jnp.dot((a4 * std).astype(bf), wstd[...], preferred_element_type=jnp.float32)
    out_ref[...] = acc + b_ref[0:1, :]


def _post0(h0, s, sq, mn, mx, cnt128, aw, wslices, b0):
    bm = 256
    grid = (N_NODES // bm,)
    blk = pl.BlockSpec((bm, PF), lambda i: (i, 0))
    cspec = pl.BlockSpec((bm, 128), lambda i: (i, 0))
    awspec = pl.BlockSpec((8, 128), lambda i: (0, 0))
    wspec = pl.BlockSpec((PF, PF), lambda i: (0, 0))
    bspec = pl.BlockSpec((8, PF), lambda i: (0, 0))
    return pl.pallas_call(
        _post0_body,
        grid=grid,
        in_specs=[blk, blk, blk, blk, blk, cspec, awspec] + [wspec] * 6 + [bspec],
        out_specs=blk,
        out_shape=jax.ShapeDtypeStruct((N_NODES, PF), jnp.float32),
    )(h0, s, sq, mn, mx, cnt128, aw, *wslices, b0)


def _post_chain_body(x_ref, w1, b1, w2, b2, w3, b3, w4, b4, wl, bl, out_ref, part_ref):
    m = x_ref[...]
    for w_ref, b_ref in ((w1, b1), (w2, b2), (w3, b3), (w4, b4)):
        m = jnp.maximum(m, 0.0).astype(jnp.bfloat16)
        m = jnp.dot(m, w_ref[...], preferred_element_type=jnp.float32) + b_ref[0:1, :]
    m = jnp.dot(m.astype(jnp.bfloat16), wl[...], preferred_element_type=jnp.float32) + bl[0:1, :]
    out_ref[...] = m
    part_ref[0, 0, :] = jnp.sum(m, axis=0)
    part_ref[0, 1, :] = jnp.sum(m * m, axis=0)


def _post_chain(x, ws, bs, wl, bl):
    bm = 512
    nb = N_NODES // bm
    grid = (nb,)
    blk = pl.BlockSpec((bm, PF), lambda i: (i, 0))
    wspec = pl.BlockSpec((PF, PF), lambda i: (0, 0))
    bspec = pl.BlockSpec((8, PF), lambda i: (0, 0))
    in_specs = [blk]
    args = [x]
    for w, b in zip(ws, bs):
        in_specs += [wspec, bspec]
        args += [w, b]
    in_specs += [wspec, bspec]
    args += [wl, bl]
    return pl.pallas_call(
        _post_chain_body,
        grid=grid,
        in_specs=in_specs,
        out_specs=[blk, pl.BlockSpec((1, 2, PF), lambda i: (i, 0, 0))],
        out_shape=[jax.ShapeDtypeStruct((N_NODES, PF), jnp.float32),
                   jax.ShapeDtypeStruct((nb, 2, PF), jnp.float32)],
    )(*args)


def _finale_body(x_ref, part_ref, batch_ref, g_ref, be_ref, w1, b1, w2, b2, w3, b3,
                 xf_ref, pool_ref):
    i = pl.program_id(0)
    colsum = jnp.sum(part_ref[:, 0, :], axis=0, keepdims=True)
    colsq = jnp.sum(part_ref[:, 1, :], axis=0, keepdims=True)
    mu = colsum / float(N_NODES)
    var = colsq / float(N_NODES) - mu * mu
    rstd = lax.rsqrt(var + 1e-5)
    h = (x_ref[...] - mu) * rstd * g_ref[0:1, :] + be_ref[0:1, :]
    h = jnp.maximum(h, 0.0)
    b = batch_ref[0, 0, :]
    b2d = jnp.broadcast_to(b[None, :], (N_GRAPHS, b.shape[0]))
    g2d = lax.broadcasted_iota(jnp.int32, (N_GRAPHS, b.shape[0]), 0)
    oh = (b2d == g2d).astype(jnp.float32)

    @pl.when(i == 0)
    def _():
        pool_ref[...] = jnp.zeros_like(pool_ref)

    pool_ref[...] += jnp.dot(oh, h, preferred_element_type=jnp.float32)
    f = jnp.maximum(jnp.dot(h.astype(jnp.bfloat16), w1[...], preferred_element_type=jnp.float32) + b1[0:1, :], 0.0)
    f = jnp.maximum(jnp.dot(f.astype(jnp.bfloat16), w2[...], preferred_element_type=jnp.float32) + b2[0:1, :], 0.0)
    xf_ref[...] = jnp.dot(f.astype(jnp.bfloat16), w3[...], preferred_element_type=jnp.float32) + b3[0:1, :]


def _finale(x, part, batch3, gamma, beta, w1, b1, w2, b2, w3, b3):
    bm = 512
    nb = N_NODES // bm
    grid = (nb,)
    blk = pl.BlockSpec((bm, PF), lambda i: (i, 0))
    return pl.pallas_call(
        _finale_body,
        grid=grid,
        in_specs=[blk,
                  pl.BlockSpec((nb, 2, PF), lambda i: (0, 0, 0)),
                  pl.BlockSpec((1, 1, bm), lambda i: (i, 0, 0)),
                  pl.BlockSpec((8, PF), lambda i: (0, 0)),
                  pl.BlockSpec((8, PF), lambda i: (0, 0)),
                  pl.BlockSpec((PF, 640), lambda i: (0, 0)),
                  pl.BlockSpec((8, 640), lambda i: (0, 0)),
                  pl.BlockSpec((640, 128), lambda i: (0, 0)),
                  pl.BlockSpec((8, 128), lambda i: (0, 0)),
                  pl.BlockSpec((128, 128), lambda i: (0, 0)),
                  pl.BlockSpec((8, 128), lambda i: (0, 0))],
        out_specs=[pl.BlockSpec((bm, 128), lambda i: (i, 0)),
                   pl.BlockSpec((N_GRAPHS, PF), lambda i: (0, 0))],
        out_shape=[jax.ShapeDtypeStruct((N_NODES, 128), jnp.float32),
                   jax.ShapeDtypeStruct((N_GRAPHS, PF), jnp.float32)],
    )(x, part, batch3, gamma, beta, w1, b1, w2, b2, w3, b3)


def _mlp2_body(p_ref, w1, b1, w2, b2, w3, b3, out_ref):
    t = jnp.maximum(jnp.dot(p_ref[...], w1[...], preferred_element_type=jnp.float32) + b1[0:1, :], 0.0)
    t = jnp.maximum(jnp.dot(t, w2[...], preferred_element_type=jnp.float32) + b2[0:1, :], 0.0)
    out_ref[...] = jnp.dot(t, w3[...], preferred_element_type=jnp.float32) + b3[0:1, :]


def _mlp2(pool, w1, b1, w2, b2, w3, b3):
    return pl.pallas_call(
        _mlp2_body,
        out_shape=jax.ShapeDtypeStruct((N_GRAPHS, 128), jnp.float32),
    )(pool, w1, b1, w2, b2, w3, b3)


_NW = 32
_EPW = N_EDGES // _NW
_GB = 32


def _gather_sc_body(hd_hbm, hs_hbm, dst_hbm, src_hbm, out_hbm,
                    dsti, srci, bufa, bufb, sema, semb):
    wid = lax.axis_index("s") * 2 + lax.axis_index("c")
    base = wid * _EPW
    pltpu.sync_copy(dst_hbm.at[pl.ds(base, _EPW)], dsti)
    pltpu.sync_copy(src_hbm.at[pl.ds(base, _EPW)], srci)

    def batch_body(b, carry):
        cpa = pltpu.async_copy(hd_hbm.at[dsti.at[pl.ds(b * _GB, _GB)]], bufa, sema)
        cpb = pltpu.async_copy(hs_hbm.at[srci.at[pl.ds(b * _GB, _GB)]], bufb, semb)
        cpa.wait()
        cpb.wait()

        def row_body(r, c2):
            for j in range(PF // 16):
                sl = pl.ds(j * 16, 16)
                bufa[r, sl] = bufa[r, sl] + bufb[r, sl]
            return c2

        lax.fori_loop(0, _GB, row_body, 0)
        pltpu.sync_copy(bufa, out_hbm.at[pl.ds(base + b * _GB, _GB)])
        return carry

    lax.fori_loop(0, _EPW // _GB, batch_body, 0)


def _edge_gather(hd, hs, dst, src):
    mesh = plsc.VectorSubcoreMesh(core_axis_name="c", subcore_axis_name="s")
    f = functools.partial(
        pl.kernel,
        out_type=jax.ShapeDtypeStruct((N_EDGES, PF), jnp.float32),
        mesh=mesh,
        scratch_types=[
            pltpu.VMEM((_EPW,), jnp.int32),
            pltpu.VMEM((_EPW,), jnp.int32),
            pltpu.VMEM((_GB, PF), jnp.float32),
            pltpu.VMEM((_GB, PF), jnp.float32),
            pltpu.SemaphoreType.DMA,
            pltpu.SemaphoreType.DMA,
        ],
    )(_gather_sc_body)
    return f(hd, hs, dst, src)


def _aggregate(m4, dst):
    s = jax.ops.segment_sum(m4, dst, num_segments=N_NODES)
    sq = jax.ops.segment_sum(m4 * m4, dst, num_segments=N_NODES)
    mn = jax.ops.segment_min(m4, dst, num_segments=N_NODES)
    mx = jax.ops.segment_max(m4, dst, num_segments=N_NODES)
    cnt = jax.ops.segment_sum(jnp.ones((N_EDGES,), jnp.float32), dst, num_segments=N_NODES)
    return s, sq, mn, mx, cnt


def kernel(x, edge_index, edge_attr, batch, params):
    f32 = jnp.float32
    xp = _pad2(x, N_NODES, PF).astype(jnp.bfloat16)
    p = params
    w1 = _pad2(p["mlp1"]["w"], PF, PF)
    b1 = _padb(p["mlp1"]["b"], PF)
    pre0w = p["pre"][0]["w"]
    wd = _pad2(pre0w[:F], PF, PF)
    bd = _padb(p["pre"][0]["b"], PF)
    ws_ = _pad2(pre0w[F:2 * F], PF, PF)
    emb = _pad2(p["edge_emb"], 32, 128)
    wenc = _pad2(p["edge_enc"]["w"], 128, PF)
    benc = _padb(p["edge_enc"]["b"], PF)
    wc = _pad2(pre0w[2 * F:], PF, PF)
    prew = [_pad2(p["pre"][i]["w"], PF, PF) for i in range(1, 5)]
    preb = [_padb(p["pre"][i]["b"], PF) for i in range(1, 5)]
    post0w = p["post"][0]["w"]
    wslices = [_pad2(post0w[i * F:(i + 1) * F], PF, PF) for i in range(6)]
    b0 = _padb(p["post"][0]["b"], PF)
    postw = [_pad2(p["post"][i]["w"], PF, PF) for i in range(1, 5)]
    postb = [_padb(p["post"][i]["b"], PF) for i in range(1, 5)]
    wl = _pad2(p["lin"]["w"], PF, PF)
    bl = _padb(p["lin"]["b"], PF)
    gamma = _padb(p["bn_gamma"], PF)
    beta = _padb(p["bn_beta"], PF)
    m2w1 = _pad2(p["mlp2"][0]["w"], PF, 640)
    m2b1 = _padb(p["mlp2"][0]["b"], 640)
    m2w2 = _pad2(p["mlp2"][1]["w"], 640, 128)
    m2b2 = _padb(p["mlp2"][1]["b"], 128)
    m2w3 = _pad2(p["mlp2"][2]["w"], 128, 128)
    m2b3 = _padb(p["mlp2"][2]["b"], 128)
    m3w1 = _pad2(p["mlp3"][0]["w"], PF, 640)
    m3b1 = _padb(p["mlp3"][0]["b"], 640)
    m3w2 = _pad2(p["mlp3"][1]["w"], 640, 128)
    m3b2 = _padb(p["mlp3"][1]["b"], 128)
    m3w3 = _pad2(p["mlp3"][2]["w"], 128, 128)
    m3b3 = _padb(p["mlp3"][2]["b"], 128)
    aw5 = jax.nn.softmax(p["agg_w"])
    aw = jnp.zeros((8, 128), f32).at[0, :5].set(aw5)

    dst = edge_index[1]
    src = edge_index[0]
    attr3 = edge_attr.astype(jnp.int32).reshape(N_EDGES // 768, 1, 768)
    batch3 = batch.astype(jnp.int32).reshape(N_NODES // 512, 1, 512)

    bf = jnp.bfloat16
    w1 = w1.astype(bf)
    wd = wd.astype(bf)
    ws_ = ws_.astype(bf)
    prew = [w.astype(bf) for w in prew]
    wslices = [w.astype(bf) for w in wslices]
    postw = [w.astype(bf) for w in postw]
    wl = wl.astype(bf)
    m3w1 = m3w1.astype(bf)
    m3w2 = m3w2.astype(bf)
    m3w3 = m3w3.astype(bf)
    ctab = _ctab(emb, wenc, benc, wc).astype(bf)
    h0, hd, hs = _node_pre(xp, w1, b1, wd, bd, ws_)
    m0 = _edge_gather(hd, hs, dst, src)
    m4 = _pre_chain(m0, attr3, ctab, prew, preb)
    s, sq, mn, mx, cnt = _aggregate(m4, dst)
    cnt128 = jnp.broadcast_to(cnt[:, None], (N_NODES, 128))
    o0 = _post0(h0, s, sq, mn, mx, cnt128, aw, wslices, b0)
    out, part = _post_chain(o0, postw, postb, wl, bl)
    xf_pad, pool = _finale(out, part, batch3, gamma, beta, m3w1, m3b1, m3w2, m3b2, m3w3, m3b3)
    xe_pad = _mlp2(pool, m2w1, m2b1, m2w2, m2b2, m2w3, m2b3)
    return xf_pad[:, :3], xe_pad[:, :1]


# speedup vs baseline: 1.2677x; 1.0609x over previous
import functools
import jax
import jax.numpy as jnp
from jax import lax
from jax.experimental import pallas as pl
from jax.experimental.pallas import tpu as pltpu
from jax.experimental.pallas import tpu_sc as plsc

N_NODES = 5120
N_EDGES = 15360
N_GRAPHS = 64
F = 1262
PF = 1280


def _pad2(a, r, c):
    return jnp.zeros((r, c), a.dtype).at[: a.shape[0], : a.shape[1]].set(a)


def _padb(b, c):
    z = jnp.zeros((8, c), b.dtype)
    return z.at[0, : b.shape[0]].set(b)


def _ctab_body(emb_ref, wenc_ref, benc_ref, wc_ref, out_ref):
    t = jnp.dot(emb_ref[...], wenc_ref[...], preferred_element_type=jnp.float32)
    t = t + benc_ref[0:1, :]
    out_ref[...] = jnp.dot(t, wc_ref[...], preferred_element_type=jnp.float32)


def _ctab(emb, wenc, benc, wc):
    return pl.pallas_call(
        _ctab_body,
        out_shape=jax.ShapeDtypeStruct((32, PF), jnp.float32),
    )(emb, wenc, benc, wc)


def _node_pre_body(x_ref, w1_ref, b1_ref, wd_ref, bd_ref, ws_ref, h0_ref, hd_ref, hs_ref):
    h0 = jnp.dot(x_ref[...], w1_ref[...], preferred_element_type=jnp.float32)
    h0 = jnp.maximum(h0 + b1_ref[0:1, :], 0.0)
    h0_ref[...] = h0
    hb = h0.astype(jnp.bfloat16)
    hd_ref[...] = jnp.dot(hb, wd_ref[...], preferred_element_type=jnp.float32) + bd_ref[0:1, :]
    hs_ref[...] = jnp.dot(hb, ws_ref[...], preferred_element_type=jnp.float32)


def _node_pre(x, w1, b1, wd, bd, ws):
    bm = 512
    grid = (N_NODES // bm,)
    blk = pl.BlockSpec((bm, PF), lambda i: (i, 0))
    wspec = pl.BlockSpec((PF, PF), lambda i: (0, 0))
    bspec = pl.BlockSpec((8, PF), lambda i: (0, 0))
    return pl.pallas_call(
        _node_pre_body,
        grid=grid,
        in_specs=[blk, wspec, bspec, wspec, bspec, wspec],
        out_specs=[blk, blk, blk],
        out_shape=[jax.ShapeDtypeStruct((N_NODES, PF), jnp.float32)] * 3,
    )(x, w1, b1, wd, bd, ws)


def _pre_chain_body(m0_ref, attr_ref, ctab_ref, w1, b1, w2, b2, w3, b3, w4, b4, out_ref):
    a = attr_ref[0, 0, :]
    oh = (a[:, None] == lax.broadcasted_iota(jnp.int32, (a.shape[0], 32), 1)).astype(jnp.bfloat16)
    m = m0_ref[...] + jnp.dot(oh, ctab_ref[...], preferred_element_type=jnp.float32)
    for w_ref, b_ref in ((w1, b1), (w2, b2), (w3, b3), (w4, b4)):
        m = jnp.maximum(m, 0.0).astype(jnp.bfloat16)
        m = jnp.dot(m, w_ref[...], preferred_element_type=jnp.float32) + b_ref[0:1, :]
    out_ref[...] = m


def _pre_chain(m0, attr3, ctab, ws, bs):
    bm = 768
    grid = (N_EDGES // bm,)
    blk = pl.BlockSpec((bm, PF), lambda i: (i, 0))
    aspec = pl.BlockSpec((1, 1, bm), lambda i: (i, 0, 0))
    cspec = pl.BlockSpec((32, PF), lambda i: (0, 0))
    wspec = pl.BlockSpec((PF, PF), lambda i: (0, 0))
    bspec = pl.BlockSpec((8, PF), lambda i: (0, 0))
    in_specs = [blk, aspec, cspec]
    args = [m0, attr3, ctab]
    for w, b in zip(ws, bs):
        in_specs += [wspec, bspec]
        args += [w, b]
    return pl.pallas_call(
        _pre_chain_body,
        grid=grid,
        in_specs=in_specs,
        out_specs=blk,
        out_shape=jax.ShapeDtypeStruct((N_EDGES, PF), jnp.float32),
    )(*args)


def _post0_body(h0_ref, s_ref, sq_ref, mn_ref, mx_ref, cnt_ref, aw_ref,
                wh, wsum, wmean, wmin, wmax, wstd, b_ref, out_ref):
    cnt = cnt_ref[...][:, 0:1]
    cnt_safe = jnp.maximum(cnt, 1.0)
    has = (cnt > 0.0).astype(jnp.float32)
    s = s_ref[...]
    mean = s / cnt_safe
    mn = mn_ref[...] * has
    mx = mx_ref[...] * has
    msq = sq_ref[...] / cnt_safe
    var = jnp.maximum(msq - mean * mean, 0.0)
    std = jnp.sqrt(var + 1e-5)
    a0 = aw_ref[0, 0]
    a1 = aw_ref[0, 1]
    a2 = aw_ref[0, 2]
    a3 = aw_ref[0, 3]
    a4 = aw_ref[0, 4]
    bf = jnp.bfloat16
    acc = jnp.dot(h0_ref[...].astype(bf), wh[...], preferred_element_type=jnp.float32)
    acc += jnp.dot((a0 * s).astype(bf), wsum[...], preferred_element_type=jnp.float32)
    acc += jnp.dot((a1 * mean).astype(bf), wmean[...], preferred_element_type=jnp.float32)
    acc += jnp.dot((a2 * mn).astype(bf), wmin[...], preferred_element_type=jnp.float32)
    acc += jnp.dot((a3 * mx).astype(bf), wmax[...], preferred_element_type=jnp.float32)
    acc += jnp.dot((a4 * std).astype(bf), wstd[...], preferred_element_type=jnp.float32)
    out_ref[...] = acc + b_ref[0:1, :]


def _post0(h0, s, sq, mn, mx, cnt128, aw, wslices, b0):
    bm = 256
    grid = (N_NODES // bm,)
    blk = pl.BlockSpec((bm, PF), lambda i: (i, 0))
    cspec = pl.BlockSpec((bm, 128), lambda i: (i, 0))
    awspec = pl.BlockSpec((8, 128), lambda i: (0, 0))
    wspec = pl.BlockSpec((PF, PF), lambda i: (0, 0))
    bspec = pl.BlockSpec((8, PF), lambda i: (0, 0))
    return pl.pallas_call(
        _post0_body,
        grid=grid,
        in_specs=[blk, blk, blk, blk, blk, cspec, awspec] + [wspec] * 6 + [bspec],
        out_specs=blk,
        out_shape=jax.ShapeDtypeStruct((N_NODES, PF), jnp.float32),
    )(h0, s, sq, mn, mx, cnt128, aw, *wslices, b0)


def _post_chain_body(x_ref, w1, b1, w2, b2, w3, b3, w4, b4, wl, bl, out_ref, part_ref):
    m = x_ref[...]
    for w_ref, b_ref in ((w1, b1), (w2, b2), (w3, b3), (w4, b4)):
        m = jnp.maximum(m, 0.0).astype(jnp.bfloat16)
        m = jnp.dot(m, w_ref[...], preferred_element_type=jnp.float32) + b_ref[0:1, :]
    m = jnp.dot(m.astype(jnp.bfloat16), wl[...], preferred_element_type=jnp.float32) + bl[0:1, :]
    out_ref[...] = m
    part_ref[0, 0, :] = jnp.sum(m, axis=0)
    part_ref[0, 1, :] = jnp.sum(m * m, axis=0)


def _post_chain(x, ws, bs, wl, bl):
    bm = 512
    nb = N_NODES // bm
    grid = (nb,)
    blk = pl.BlockSpec((bm, PF), lambda i: (i, 0))
    wspec = pl.BlockSpec((PF, PF), lambda i: (0, 0))
    bspec = pl.BlockSpec((8, PF), lambda i: (0, 0))
    in_specs = [blk]
    args = [x]
    for w, b in zip(ws, bs):
        in_specs += [wspec, bspec]
        args += [w, b]
    in_specs += [wspec, bspec]
    args += [wl, bl]
    return pl.pallas_call(
        _post_chain_body,
        grid=grid,
        in_specs=in_specs,
        out_specs=[blk, pl.BlockSpec((1, 2, PF), lambda i: (i, 0, 0))],
        out_shape=[jax.ShapeDtypeStruct((N_NODES, PF), jnp.float32),
                   jax.ShapeDtypeStruct((nb, 2, PF), jnp.float32)],
    )(*args)


def _finale_body(x_ref, part_ref, batch_ref, g_ref, be_ref, w1, b1, w2, b2, w3, b3,
                 xf_ref, pool_ref):
    i = pl.program_id(0)
    colsum = jnp.sum(part_ref[:, 0, :], axis=0, keepdims=True)
    colsq = jnp.sum(part_ref[:, 1, :], axis=0, keepdims=True)
    mu = colsum / float(N_NODES)
    var = colsq / float(N_NODES) - mu * mu
    rstd = lax.rsqrt(var + 1e-5)
    h = (x_ref[...] - mu) * rstd * g_ref[0:1, :] + be_ref[0:1, :]
    h = jnp.maximum(h, 0.0)
    b = batch_ref[0, 0, :]
    b2d = jnp.broadcast_to(b[None, :], (N_GRAPHS, b.shape[0]))
    g2d = lax.broadcasted_iota(jnp.int32, (N_GRAPHS, b.shape[0]), 0)
    oh = (b2d == g2d).astype(jnp.float32)

    @pl.when(i == 0)
    def _():
        pool_ref[...] = jnp.zeros_like(pool_ref)

    pool_ref[...] += jnp.dot(oh, h, preferred_element_type=jnp.float32)
    f = jnp.maximum(jnp.dot(h.astype(jnp.bfloat16), w1[...], preferred_element_type=jnp.float32) + b1[0:1, :], 0.0)
    f = jnp.maximum(jnp.dot(f.astype(jnp.bfloat16), w2[...], preferred_element_type=jnp.float32) + b2[0:1, :], 0.0)
    xf_ref[...] = jnp.dot(f.astype(jnp.bfloat16), w3[...], preferred_element_type=jnp.float32) + b3[0:1, :]


def _finale(x, part, batch3, gamma, beta, w1, b1, w2, b2, w3, b3):
    bm = 512
    nb = N_NODES // bm
    grid = (nb,)
    blk = pl.BlockSpec((bm, PF), lambda i: (i, 0))
    return pl.pallas_call(
        _finale_body,
        grid=grid,
        in_specs=[blk,
                  pl.BlockSpec((nb, 2, PF), lambda i: (0, 0, 0)),
                  pl.BlockSpec((1, 1, bm), lambda i: (i, 0, 0)),
                  pl.BlockSpec((8, PF), lambda i: (0, 0)),
                  pl.BlockSpec((8, PF), lambda i: (0, 0)),
                  pl.BlockSpec((PF, 640), lambda i: (0, 0)),
                  pl.BlockSpec((8, 640), lambda i: (0, 0)),
                  pl.BlockSpec((640, 128), lambda i: (0, 0)),
                  pl.BlockSpec((8, 128), lambda i: (0, 0)),
                  pl.BlockSpec((128, 128), lambda i: (0, 0)),
                  pl.BlockSpec((8, 128), lambda i: (0, 0))],
        out_specs=[pl.BlockSpec((bm, 128), lambda i: (i, 0)),
                   pl.BlockSpec((N_GRAPHS, PF), lambda i: (0, 0))],
        out_shape=[jax.ShapeDtypeStruct((N_NODES, 128), jnp.float32),
                   jax.ShapeDtypeStruct((N_GRAPHS, PF), jnp.float32)],
    )(x, part, batch3, gamma, beta, w1, b1, w2, b2, w3, b3)


def _mlp2_body(p_ref, w1, b1, w2, b2, w3, b3, out_ref):
    t = jnp.maximum(jnp.dot(p_ref[...], w1[...], preferred_element_type=jnp.float32) + b1[0:1, :], 0.0)
    t = jnp.maximum(jnp.dot(t, w2[...], preferred_element_type=jnp.float32) + b2[0:1, :], 0.0)
    out_ref[...] = jnp.dot(t, w3[...], preferred_element_type=jnp.float32) + b3[0:1, :]


def _mlp2(pool, w1, b1, w2, b2, w3, b3):
    return pl.pallas_call(
        _mlp2_body,
        out_shape=jax.ShapeDtypeStruct((N_GRAPHS, 128), jnp.float32),
    )(pool, w1, b1, w2, b2, w3, b3)


_NW = 32
_EPW = N_EDGES // _NW
_GB = 32


def _gather_sc_body(hd_hbm, hs_hbm, dst_hbm, src_hbm, out_hbm,
                    dsti, srci, bufa, bufb, sema, semb):
    wid = lax.axis_index("s") * 2 + lax.axis_index("c")
    base = wid * _EPW
    pltpu.sync_copy(dst_hbm.at[pl.ds(base, _EPW)], dsti)
    pltpu.sync_copy(src_hbm.at[pl.ds(base, _EPW)], srci)

    def batch_body(b, carry):
        cpa = pltpu.async_copy(hd_hbm.at[dsti.at[pl.ds(b * _GB, _GB)]], bufa, sema)
        cpb = pltpu.async_copy(hs_hbm.at[srci.at[pl.ds(b * _GB, _GB)]], bufb, semb)
        cpa.wait()
        cpb.wait()

        def row_body(r, c2):
            for j in range(PF // 16):
                sl = pl.ds(j * 16, 16)
                bufa[r, sl] = bufa[r, sl] + bufb[r, sl]
            return c2

        lax.fori_loop(0, _GB, row_body, 0)
        pltpu.sync_copy(bufa, out_hbm.at[pl.ds(base + b * _GB, _GB)])
        return carry

    lax.fori_loop(0, _EPW // _GB, batch_body, 0)


def _edge_gather(hd, hs, dst, src):
    mesh = plsc.VectorSubcoreMesh(core_axis_name="c", subcore_axis_name="s")
    f = functools.partial(
        pl.kernel,
        out_type=jax.ShapeDtypeStruct((N_EDGES, PF), jnp.float32),
        mesh=mesh,
        scratch_types=[
            pltpu.VMEM((_EPW,), jnp.int32),
            pltpu.VMEM((_EPW,), jnp.int32),
            pltpu.VMEM((_GB, PF), jnp.float32),
            pltpu.VMEM((_GB, PF), jnp.float32),
            pltpu.SemaphoreType.DMA,
            pltpu.SemaphoreType.DMA,
        ],
    )(_gather_sc_body)
    return f(hd, hs, dst, src)


_NPW = N_NODES // _NW
_BKN = 16
_NBK = _NPW // _BKN
_TLCAP = 1024
_BKCAP = 256
_RB = 8
_FINF = 3.0e38


def _lperm(v, idx):
    return lax.gather(
        v, idx[:, None],
        dimension_numbers=lax.GatherDimensionNumbers(
            offset_dims=(), collapsed_slice_dims=(0,), start_index_map=(0,)),
        slice_sizes=(1,), mode=lax.GatherScatterMode.PROMISE_IN_BOUNDS)


def _prefix16(mi, lane):
    p = mi
    for k in (1, 2, 4, 8):
        sh = _lperm(p, jnp.maximum(lane - k, 0))
        p = p + jnp.where(lane >= k, sh, 0)
    return p


def _agg_sc_body(m4_hbm, dst_hbm, s_hbm, q_hbm, n_hbm, x_hbm, c_hbm,
                 dstv, tle, tld, eb, lb, acc_s, acc_q, acc_n, acc_x,
                 rows, cntv, cnt2, sem):
    i32 = jnp.int32
    f32 = jnp.float32
    wid = lax.axis_index("s") * 2 + lax.axis_index("c")
    lo = wid * _NPW
    pltpu.sync_copy(dst_hbm, dstv)
    lane = lax.iota(i32, 16)
    lane0f = (lane == 0).astype(f32)
    fifteen = jnp.full((16,), 15, i32)

    def czero(i, c):
        cnt2[i, :] = jnp.zeros((16,), f32)
        return c

    lax.fori_loop(0, _NPW, czero, 0)

    def wcomp(i, nvec):
        v = dstv[pl.ds(i * 16, 16)]
        inb = (v >= lo) & (v < lo + _NPW)
        mi = inb.astype(i32)
        p = _prefix16(mi, lane)
        pos = nvec + p - mi
        posw = jnp.minimum(jnp.where(inb, pos, _TLCAP - 1), _TLCAP - 1)
        plsc.store_scatter(tle, [posw], lane + i * 16)
        plsc.store_scatter(tld, [posw], v)
        return nvec + _lperm(p, fifteen)

    nvec = lax.fori_loop(0, N_EDGES // 16, wcomp, jnp.zeros((16,), i32))
    n_t = jnp.minimum(nvec[0], _TLCAP - 1)

    def bucket(k, carry):
        blo = lo + k * _BKN

        def initrow(r, c):
            for j in range(PF // 16):
                sl = pl.ds(j * 16, 16)
                acc_s[r, sl] = jnp.zeros((16,), f32)
                acc_q[r, sl] = jnp.zeros((16,), f32)
                acc_n[r, sl] = jnp.full((16,), _FINF, f32)
                acc_x[r, sl] = jnp.full((16,), -_FINF, f32)
            return c

        lax.fori_loop(0, _BKN, initrow, 0)

        def ezero(i, c):
            eb[pl.ds(i * 16, 16)] = jnp.zeros((16,), i32)
            return c

        lax.fori_loop(0, _BKCAP // 16, ezero, 0)

        def bcomp(i, nbv):
            v = tld[pl.ds(i * 16, 16)]
            e = tle[pl.ds(i * 16, 16)]
            valid = (lane + i * 16) < n_t
            msk = valid & (v >= blo) & (v < blo + _BKN)
            mi = msk.astype(i32)
            p = _prefix16(mi, lane)
            pos = nbv + p - mi
            posw = jnp.minimum(jnp.where(msk, pos, _BKCAP - 1), _BKCAP - 1)
            plsc.store_scatter(eb, [posw], e)
            plsc.store_scatter(lb, [posw], v - blo)
            return nbv + _lperm(p, fifteen)

        nbv = lax.fori_loop(0, (n_t + 15) // 16, bcomp, jnp.zeros((16,), i32))
        n_b = jnp.minimum(nbv[0], _BKCAP - 16)

        def ebatch(bi, c):
            pltpu.async_copy(m4_hbm.at[eb.at[pl.ds(bi * _RB, _RB)]], rows, sem).wait()

            def erow(r, c2):
                pidx = jnp.full((16,), bi * _RB + r, i32)
                l = plsc.load_gather(lb, [pidx])[0]
                ci = k * _BKN + l
                cnt2[ci, :] = cnt2[ci, :] + lane0f
                for j in range(PF // 16):
                    sl = pl.ds(j * 16, 16)
                    mv = rows[r, sl]
                    acc_s[l, sl] = acc_s[l, sl] + mv
                    acc_q[l, sl] = acc_q[l, sl] + mv * mv
                    acc_n[l, sl] = jnp.minimum(acc_n[l, sl], mv)
                    acc_x[l, sl] = jnp.maximum(acc_x[l, sl], mv)
                return c2

            lax.fori_loop(0, jnp.minimum(_RB, n_b - bi * _RB), erow, 0)
            return c

        lax.fori_loop(0, (n_b + _RB - 1) // _RB, ebatch, 0)

        pltpu.sync_copy(acc_s, s_hbm.at[pl.ds(blo, _BKN)])
        pltpu.sync_copy(acc_q, q_hbm.at[pl.ds(blo, _BKN)])
        pltpu.sync_copy(acc_n, n_hbm.at[pl.ds(blo, _BKN)])
        pltpu.sync_copy(acc_x, x_hbm.at[pl.ds(blo, _BKN)])
        return carry

    lax.fori_loop(0, _NBK, bucket, 0)

    def cgath(g, c):
        ridx = lane + g * 16
        zidx = jnp.zeros((16,), i32)
        cntv[pl.ds(g * 16, 16)] = plsc.load_gather(cnt2, [ridx, zidx])
        return c

    lax.fori_loop(0, _NPW // 16, cgath, 0)
    pltpu.sync_copy(cntv, c_hbm.at[pl.ds(lo, _NPW)])


def _aggregate(m4, dst):
    mesh = plsc.VectorSubcoreMesh(core_axis_name="c", subcore_axis_name="s")
    f = functools.partial(
        pl.kernel,
        out_type=[jax.ShapeDtypeStruct((N_NODES, PF), jnp.float32)] * 4
        + [jax.ShapeDtypeStruct((N_NODES,), jnp.float32)],
        mesh=mesh,
        compiler_params=pltpu.CompilerParams(needs_layout_passes=False),
        scratch_types=[
            pltpu.VMEM((N_EDGES,), jnp.int32),
            pltpu.VMEM((_TLCAP,), jnp.int32),
            pltpu.VMEM((_TLCAP,), jnp.int32),
            pltpu.VMEM((_BKCAP,), jnp.int32),
            pltpu.VMEM((_BKCAP + 16,), jnp.int32),
            pltpu.VMEM((_BKN, PF), jnp.float32),
            pltpu.VMEM((_BKN, PF), jnp.float32),
            pltpu.VMEM((_BKN, PF), jnp.float32),
            pltpu.VMEM((_BKN, PF), jnp.float32),
            pltpu.VMEM((_RB, PF), jnp.float32),
            pltpu.VMEM((_NPW,), jnp.float32),
            pltpu.VMEM((_NPW, 16), jnp.float32),
            pltpu.SemaphoreType.DMA,
        ],
    )(_agg_sc_body)
    s, sq, mn, mx, cnt = f(m4, dst)
    return s, sq, mn, mx, cnt


def kernel(x, edge_index, edge_attr, batch, params):
    f32 = jnp.float32
    xp = _pad2(x, N_NODES, PF).astype(jnp.bfloat16)
    p = params
    w1 = _pad2(p["mlp1"]["w"], PF, PF)
    b1 = _padb(p["mlp1"]["b"], PF)
    pre0w = p["pre"][0]["w"]
    wd = _pad2(pre0w[:F], PF, PF)
    bd = _padb(p["pre"][0]["b"], PF)
    ws_ = _pad2(pre0w[F:2 * F], PF, PF)
    emb = _pad2(p["edge_emb"], 32, 128)
    wenc = _pad2(p["edge_enc"]["w"], 128, PF)
    benc = _padb(p["edge_enc"]["b"], PF)
    wc = _pad2(pre0w[2 * F:], PF, PF)
    prew = [_pad2(p["pre"][i]["w"], PF, PF) for i in range(1, 5)]
    preb = [_padb(p["pre"][i]["b"], PF) for i in range(1, 5)]
    post0w = p["post"][0]["w"]
    wslices = [_pad2(post0w[i * F:(i + 1) * F], PF, PF) for i in range(6)]
    b0 = _padb(p["post"][0]["b"], PF)
    postw = [_pad2(p["post"][i]["w"], PF, PF) for i in range(1, 5)]
    postb = [_padb(p["post"][i]["b"], PF) for i in range(1, 5)]
    wl = _pad2(p["lin"]["w"], PF, PF)
    bl = _padb(p["lin"]["b"], PF)
    gamma = _padb(p["bn_gamma"], PF)
    beta = _padb(p["bn_beta"], PF)
    m2w1 = _pad2(p["mlp2"][0]["w"], PF, 640)
    m2b1 = _padb(p["mlp2"][0]["b"], 640)
    m2w2 = _pad2(p["mlp2"][1]["w"], 640, 128)
    m2b2 = _padb(p["mlp2"][1]["b"], 128)
    m2w3 = _pad2(p["mlp2"][2]["w"], 128, 128)
    m2b3 = _padb(p["mlp2"][2]["b"], 128)
    m3w1 = _pad2(p["mlp3"][0]["w"], PF, 640)
    m3b1 = _padb(p["mlp3"][0]["b"], 640)
    m3w2 = _pad2(p["mlp3"][1]["w"], 640, 128)
    m3b2 = _padb(p["mlp3"][1]["b"], 128)
    m3w3 = _pad2(p["mlp3"][2]["w"], 128, 128)
    m3b3 = _padb(p["mlp3"][2]["b"], 128)
    aw5 = jax.nn.softmax(p["agg_w"])
    aw = jnp.zeros((8, 128), f32).at[0, :5].set(aw5)

    dst = edge_index[1]
    src = edge_index[0]
    attr3 = edge_attr.astype(jnp.int32).reshape(N_EDGES // 768, 1, 768)
    batch3 = batch.astype(jnp.int32).reshape(N_NODES // 512, 1, 512)

    bf = jnp.bfloat16
    w1 = w1.astype(bf)
    wd = wd.astype(bf)
    ws_ = ws_.astype(bf)
    prew = [w.astype(bf) for w in prew]
    wslices = [w.astype(bf) for w in wslices]
    postw = [w.astype(bf) for w in postw]
    wl = wl.astype(bf)
    m3w1 = m3w1.astype(bf)
    m3w2 = m3w2.astype(bf)
    m3w3 = m3w3.astype(bf)
    ctab = _ctab(emb, wenc, benc, wc).astype(bf)
    h0, hd, hs = _node_pre(xp, w1, b1, wd, bd, ws_)
    m0 = _edge_gather(hd, hs, dst, src)
    m4 = _pre_chain(m0, attr3, ctab, prew, preb)
    s, sq, mn, mx, cnt = _aggregate(m4, dst)
    cnt128 = jnp.broadcast_to(cnt[:, None], (N_NODES, 128))
    o0 = _post0(h0, s, sq, mn, mx, cnt128, aw, wslices, b0)
    out, part = _post_chain(o0, postw, postb, wl, bl)
    xf_pad, pool = _finale(out, part, batch3, gamma, beta, m3w1, m3b1, m3w2, m3b2, m3w3, m3b3)
    xe_pad = _mlp2(pool, m2w1, m2b1, m2w2, m2b2, m2w3, m2b3)
    return xf_pad[:, :3], xe_pad[:, :1]


# speedup vs baseline: 1.2841x; 1.0130x over previous
import functools
import jax
import jax.numpy as jnp
from jax import lax
from jax.experimental import pallas as pl
from jax.experimental.pallas import tpu as pltpu
from jax.experimental.pallas import tpu_sc as plsc

N_NODES = 5120
N_EDGES = 15360
N_GRAPHS = 64
F = 1262
PF = 1280


def _pad2(a, r, c):
    return jnp.zeros((r, c), a.dtype).at[: a.shape[0], : a.shape[1]].set(a)


def _padb(b, c):
    z = jnp.zeros((8, c), b.dtype)
    return z.at[0, : b.shape[0]].set(b)


def _ctab_body(emb_ref, wenc_ref, benc_ref, wc_ref, out_ref):
    t = jnp.dot(emb_ref[...], wenc_ref[...], preferred_element_type=jnp.float32)
    t = t + benc_ref[0:1, :]
    out_ref[...] = jnp.dot(t, wc_ref[...], preferred_element_type=jnp.float32)


def _ctab(emb, wenc, benc, wc):
    return pl.pallas_call(
        _ctab_body,
        out_shape=jax.ShapeDtypeStruct((32, PF), jnp.float32),
    )(emb, wenc, benc, wc)


def _node_pre_body(x_ref, w1_ref, b1_ref, wd_ref, bd_ref, ws_ref, h0_ref, hd_ref, hs_ref):
    h0 = jnp.dot(x_ref[...], w1_ref[...], preferred_element_type=jnp.float32)
    h0 = jnp.maximum(h0 + b1_ref[0:1, :], 0.0)
    h0_ref[...] = h0
    hb = h0.astype(jnp.bfloat16)
    hd_ref[...] = jnp.dot(hb, wd_ref[...], preferred_element_type=jnp.float32) + bd_ref[0:1, :]
    hs_ref[...] = jnp.dot(hb, ws_ref[...], preferred_element_type=jnp.float32)


def _node_pre(x, w1, b1, wd, bd, ws):
    bm = 512
    grid = (N_NODES // bm,)
    blk = pl.BlockSpec((bm, PF), lambda i: (i, 0))
    wspec = pl.BlockSpec((PF, PF), lambda i: (0, 0))
    bspec = pl.BlockSpec((8, PF), lambda i: (0, 0))
    return pl.pallas_call(
        _node_pre_body,
        grid=grid,
        in_specs=[blk, wspec, bspec, wspec, bspec, wspec],
        out_specs=[blk, blk, blk],
        out_shape=[jax.ShapeDtypeStruct((N_NODES, PF), jnp.float32)] * 3,
    )(x, w1, b1, wd, bd, ws)


def _pre_chain_body(m0_ref, attr_ref, ctab_ref, w1, b1, w2, b2, w3, b3, w4, b4, out_ref):
    a = attr_ref[0, 0, :]
    oh = (a[:, None] == lax.broadcasted_iota(jnp.int32, (a.shape[0], 32), 1)).astype(jnp.bfloat16)
    m = m0_ref[...] + jnp.dot(oh, ctab_ref[...], preferred_element_type=jnp.float32)
    for w_ref, b_ref in ((w1, b1), (w2, b2), (w3, b3), (w4, b4)):
        m = jnp.maximum(m, 0.0).astype(jnp.bfloat16)
        m = jnp.dot(m, w_ref[...], preferred_element_type=jnp.float32) + b_ref[0:1, :]
    out_ref[...] = m


def _pre_chain(m0, attr3, ctab, ws, bs):
    bm = 768
    grid = (N_EDGES // bm,)
    blk = pl.BlockSpec((bm, PF), lambda i: (i, 0))
    aspec = pl.BlockSpec((1, 1, bm), lambda i: (i, 0, 0))
    cspec = pl.BlockSpec((32, PF), lambda i: (0, 0))
    wspec = pl.BlockSpec((PF, PF), lambda i: (0, 0))
    bspec = pl.BlockSpec((8, PF), lambda i: (0, 0))
    in_specs = [blk, aspec, cspec]
    args = [m0, attr3, ctab]
    for w, b in zip(ws, bs):
        in_specs += [wspec, bspec]
        args += [w, b]
    return pl.pallas_call(
        _pre_chain_body,
        grid=grid,
        in_specs=in_specs,
        out_specs=blk,
        out_shape=jax.ShapeDtypeStruct((N_EDGES, PF), jnp.float32),
    )(*args)


def _post0_body(h0_ref, s_ref, sq_ref, mn_ref, mx_ref, cnt_ref, aw_ref,
                wh, wsum, wmean, wmin, wmax, wstd, b_ref, out_ref):
    cnt = cnt_ref[...][:, 0:1]
    cnt_safe = jnp.maximum(cnt, 1.0)
    has = (cnt > 0.0).astype(jnp.float32)
    s = s_ref[...]
    mean = s / cnt_safe
    mn = mn_ref[...] * has
    mx = mx_ref[...] * has
    msq = sq_ref[...] / cnt_safe
    var = jnp.maximum(msq - mean * mean, 0.0)
    std = jnp.sqrt(var + 1e-5)
    a0 = aw_ref[0, 0]
    a1 = aw_ref[0, 1]
    a2 = aw_ref[0, 2]
    a3 = aw_ref[0, 3]
    a4 = aw_ref[0, 4]
    bf = jnp.bfloat16
    acc = jnp.dot(h0_ref[...].astype(bf), wh[...], preferred_element_type=jnp.float32)
    acc += jnp.dot((a0 * s).astype(bf), wsum[...], preferred_element_type=jnp.float32)
    acc += jnp.dot((a1 * mean).astype(bf), wmean[...], preferred_element_type=jnp.float32)
    acc += jnp.dot((a2 * mn).astype(bf), wmin[...], preferred_element_type=jnp.float32)
    acc += jnp.dot((a3 * mx).astype(bf), wmax[...], preferred_element_type=jnp.float32)
    acc += jnp.dot((a4 * std).astype(bf), wstd[...], preferred_element_type=jnp.float32)
    out_ref[...] = acc + b_ref[0:1, :]


def _post0(h0, s, sq, mn, mx, cnt128, aw, wslices, b0):
    bm = 256
    grid = (N_NODES // bm,)
    blk = pl.BlockSpec((bm, PF), lambda i: (i, 0))
    cspec = pl.BlockSpec((bm, 128), lambda i: (i, 0))
    awspec = pl.BlockSpec((8, 128), lambda i: (0, 0))
    wspec = pl.BlockSpec((PF, PF), lambda i: (0, 0))
    bspec = pl.BlockSpec((8, PF), lambda i: (0, 0))
    return pl.pallas_call(
        _post0_body,
        grid=grid,
        in_specs=[blk, blk, blk, blk, blk, cspec, awspec] + [wspec] * 6 + [bspec],
        out_specs=blk,
        out_shape=jax.ShapeDtypeStruct((N_NODES, PF), jnp.float32),
    )(h0, s, sq, mn, mx, cnt128, aw, *wslices, b0)


def _post_chain_body(x_ref, w1, b1, w2, b2, w3, b3, w4, b4, wl, bl, out_ref, part_ref):
    m = x_ref[...]
    for w_ref, b_ref in ((w1, b1), (w2, b2), (w3, b3), (w4, b4)):
        m = jnp.maximum(m, 0.0).astype(jnp.bfloat16)
        m = jnp.dot(m, w_ref[...], preferred_element_type=jnp.float32) + b_ref[0:1, :]
    m = jnp.dot(m.astype(jnp.bfloat16), wl[...], preferred_element_type=jnp.float32) + bl[0:1, :]
    out_ref[...] = m
    part_ref[0, 0, :] = jnp.sum(m, axis=0)
    part_ref[0, 1, :] = jnp.sum(m * m, axis=0)


def _post_chain(x, ws, bs, wl, bl):
    bm = 512
    nb = N_NODES // bm
    grid = (nb,)
    blk = pl.BlockSpec((bm, PF), lambda i: (i, 0))
    wspec = pl.BlockSpec((PF, PF), lambda i: (0, 0))
    bspec = pl.BlockSpec((8, PF), lambda i: (0, 0))
    in_specs = [blk]
    args = [x]
    for w, b in zip(ws, bs):
        in_specs += [wspec, bspec]
        args += [w, b]
    in_specs += [wspec, bspec]
    args += [wl, bl]
    return pl.pallas_call(
        _post_chain_body,
        grid=grid,
        in_specs=in_specs,
        out_specs=[blk, pl.BlockSpec((1, 2, PF), lambda i: (i, 0, 0))],
        out_shape=[jax.ShapeDtypeStruct((N_NODES, PF), jnp.float32),
                   jax.ShapeDtypeStruct((nb, 2, PF), jnp.float32)],
    )(*args)


def _finale_body(x_ref, part_ref, batch_ref, g_ref, be_ref, w1, b1, w2, b2, w3, b3,
                 xf_ref, pool_ref):
    i = pl.program_id(0)
    colsum = jnp.sum(part_ref[:, 0, :], axis=0, keepdims=True)
    colsq = jnp.sum(part_ref[:, 1, :], axis=0, keepdims=True)
    mu = colsum / float(N_NODES)
    var = colsq / float(N_NODES) - mu * mu
    rstd = lax.rsqrt(var + 1e-5)
    h = (x_ref[...] - mu) * rstd * g_ref[0:1, :] + be_ref[0:1, :]
    h = jnp.maximum(h, 0.0)
    b = batch_ref[0, 0, :]
    b2d = jnp.broadcast_to(b[None, :], (N_GRAPHS, b.shape[0]))
    g2d = lax.broadcasted_iota(jnp.int32, (N_GRAPHS, b.shape[0]), 0)
    oh = (b2d == g2d).astype(jnp.float32)

    @pl.when(i == 0)
    def _():
        pool_ref[...] = jnp.zeros_like(pool_ref)

    pool_ref[...] += jnp.dot(oh, h, preferred_element_type=jnp.float32)
    f = jnp.maximum(jnp.dot(h.astype(jnp.bfloat16), w1[...], preferred_element_type=jnp.float32) + b1[0:1, :], 0.0)
    f = jnp.maximum(jnp.dot(f.astype(jnp.bfloat16), w2[...], preferred_element_type=jnp.float32) + b2[0:1, :], 0.0)
    xf_ref[...] = jnp.dot(f.astype(jnp.bfloat16), w3[...], preferred_element_type=jnp.float32) + b3[0:1, :]


def _finale(x, part, batch3, gamma, beta, w1, b1, w2, b2, w3, b3):
    bm = 512
    nb = N_NODES // bm
    grid = (nb,)
    blk = pl.BlockSpec((bm, PF), lambda i: (i, 0))
    return pl.pallas_call(
        _finale_body,
        grid=grid,
        in_specs=[blk,
                  pl.BlockSpec((nb, 2, PF), lambda i: (0, 0, 0)),
                  pl.BlockSpec((1, 1, bm), lambda i: (i, 0, 0)),
                  pl.BlockSpec((8, PF), lambda i: (0, 0)),
                  pl.BlockSpec((8, PF), lambda i: (0, 0)),
                  pl.BlockSpec((PF, 640), lambda i: (0, 0)),
                  pl.BlockSpec((8, 640), lambda i: (0, 0)),
                  pl.BlockSpec((640, 128), lambda i: (0, 0)),
                  pl.BlockSpec((8, 128), lambda i: (0, 0)),
                  pl.BlockSpec((128, 128), lambda i: (0, 0)),
                  pl.BlockSpec((8, 128), lambda i: (0, 0))],
        out_specs=[pl.BlockSpec((bm, 128), lambda i: (i, 0)),
                   pl.BlockSpec((N_GRAPHS, PF), lambda i: (0, 0))],
        out_shape=[jax.ShapeDtypeStruct((N_NODES, 128), jnp.float32),
                   jax.ShapeDtypeStruct((N_GRAPHS, PF), jnp.float32)],
    )(x, part, batch3, gamma, beta, w1, b1, w2, b2, w3, b3)


def _mlp2_body(p_ref, w1, b1, w2, b2, w3, b3, out_ref):
    t = jnp.maximum(jnp.dot(p_ref[...], w1[...], preferred_element_type=jnp.float32) + b1[0:1, :], 0.0)
    t = jnp.maximum(jnp.dot(t, w2[...], preferred_element_type=jnp.float32) + b2[0:1, :], 0.0)
    out_ref[...] = jnp.dot(t, w3[...], preferred_element_type=jnp.float32) + b3[0:1, :]


def _mlp2(pool, w1, b1, w2, b2, w3, b3):
    return pl.pallas_call(
        _mlp2_body,
        out_shape=jax.ShapeDtypeStruct((N_GRAPHS, 128), jnp.float32),
    )(pool, w1, b1, w2, b2, w3, b3)


_NW = 32
_EPW = N_EDGES // _NW
_GB = 24


def _gather_sc_body(hd_hbm, hs_hbm, dst_hbm, src_hbm, out_hbm,
                    dsti, srci, bufa0, bufa1, bufb0, bufb1,
                    sa0, sa1, sb0, sb1, so0, so1):
    wid = lax.axis_index("s") * 2 + lax.axis_index("c")
    base = wid * _EPW
    pltpu.sync_copy(dst_hbm.at[pl.ds(base, _EPW)], dsti)
    pltpu.sync_copy(src_hbm.at[pl.ds(base, _EPW)], srci)
    bufa = [bufa0, bufa1]
    bufb = [bufb0, bufb1]
    sa = [sa0, sa1]
    sb = [sb0, sb1]
    so = [so0, so1]
    nb = _EPW // _GB
    outh = [None, None]

    def issue(b):
        sl = b % 2
        if outh[sl] is not None:
            outh[sl].wait()
            outh[sl] = None
        ca = pltpu.async_copy(hd_hbm.at[dsti.at[pl.ds(b * _GB, _GB)]], bufa[sl], sa[sl])
        cb = pltpu.async_copy(hs_hbm.at[srci.at[pl.ds(b * _GB, _GB)]], bufb[sl], sb[sl])
        return ca, cb

    pend = issue(0)
    for b in range(nb):
        sl = b % 2
        nxt = issue(b + 1) if b + 1 < nb else None
        pend[0].wait()
        pend[1].wait()
        ba = bufa[sl]
        bb = bufb[sl]

        def row_body(r, c2, ba=ba, bb=bb):
            for j in range(PF // 16):
                slc = pl.ds(j * 16, 16)
                ba[r, slc] = ba[r, slc] + bb[r, slc]
            return c2

        lax.fori_loop(0, _GB, row_body, 0)
        outh[sl] = pltpu.async_copy(ba, out_hbm.at[pl.ds(base + b * _GB, _GB)], so[sl])
        pend = nxt
    if outh[0] is not None:
        outh[0].wait()
    if outh[1] is not None:
        outh[1].wait()


def _edge_gather(hd, hs, dst, src):
    mesh = plsc.VectorSubcoreMesh(core_axis_name="c", subcore_axis_name="s")
    f = functools.partial(
        pl.kernel,
        out_type=jax.ShapeDtypeStruct((N_EDGES, PF), jnp.float32),
        mesh=mesh,
        scratch_types=[
            pltpu.VMEM((_EPW,), jnp.int32),
            pltpu.VMEM((_EPW,), jnp.int32),
            pltpu.VMEM((_GB, PF), jnp.float32),
            pltpu.VMEM((_GB, PF), jnp.float32),
            pltpu.VMEM((_GB, PF), jnp.float32),
            pltpu.VMEM((_GB, PF), jnp.float32),
            pltpu.SemaphoreType.DMA,
            pltpu.SemaphoreType.DMA,
            pltpu.SemaphoreType.DMA,
            pltpu.SemaphoreType.DMA,
            pltpu.SemaphoreType.DMA,
            pltpu.SemaphoreType.DMA,
        ],
    )(_gather_sc_body)
    return f(hd, hs, dst, src)


_NPW = N_NODES // _NW
_BKN = 16
_NBK = _NPW // _BKN
_TLCAP = 1024
_BKCAP = 256
_RB = 8
_FINF = 3.0e38


def _lperm(v, idx):
    return lax.gather(
        v, idx[:, None],
        dimension_numbers=lax.GatherDimensionNumbers(
            offset_dims=(), collapsed_slice_dims=(0,), start_index_map=(0,)),
        slice_sizes=(1,), mode=lax.GatherScatterMode.PROMISE_IN_BOUNDS)


def _prefix16(mi, lane):
    p = mi
    for k in (1, 2, 4, 8):
        sh = _lperm(p, jnp.maximum(lane - k, 0))
        p = p + jnp.where(lane >= k, sh, 0)
    return p


def _agg_sc_body(m4_hbm, dst_hbm, s_hbm, q_hbm, n_hbm, x_hbm, c_hbm,
                 dstv, tle, tld, eb, lb, acc_s, acc_q, acc_n, acc_x,
                 rows, cntv, cnt2, sem):
    i32 = jnp.int32
    f32 = jnp.float32
    wid = lax.axis_index("s") * 2 + lax.axis_index("c")
    lo = wid * _NPW
    pltpu.sync_copy(dst_hbm, dstv)
    lane = lax.iota(i32, 16)
    lane0f = (lane == 0).astype(f32)
    fifteen = jnp.full((16,), 15, i32)

    def czero(i, c):
        cnt2[i, :] = jnp.zeros((16,), f32)
        return c

    lax.fori_loop(0, _NPW, czero, 0)

    def wcomp(i, nvec):
        v = dstv[pl.ds(i * 16, 16)]
        inb = (v >= lo) & (v < lo + _NPW)
        mi = inb.astype(i32)
        p = _prefix16(mi, lane)
        pos = nvec + p - mi
        posw = jnp.minimum(jnp.where(inb, pos, _TLCAP - 1), _TLCAP - 1)
        plsc.store_scatter(tle, [posw], lane + i * 16)
        plsc.store_scatter(tld, [posw], v)
        return nvec + _lperm(p, fifteen)

    nvec = lax.fori_loop(0, N_EDGES // 16, wcomp, jnp.zeros((16,), i32))
    n_t = jnp.minimum(nvec[0], _TLCAP - 1)

    def bucket(k, carry):
        blo = lo + k * _BKN

        def initrow(r, c):
            for j in range(PF // 16):
                sl = pl.ds(j * 16, 16)
                acc_s[r, sl] = jnp.zeros((16,), f32)
                acc_q[r, sl] = jnp.zeros((16,), f32)
                acc_n[r, sl] = jnp.full((16,), _FINF, f32)
                acc_x[r, sl] = jnp.full((16,), -_FINF, f32)
            return c

        lax.fori_loop(0, _BKN, initrow, 0)

        def ezero(i, c):
            eb[pl.ds(i * 16, 16)] = jnp.zeros((16,), i32)
            return c

        lax.fori_loop(0, _BKCAP // 16, ezero, 0)

        def bcomp(i, nbv):
            v = tld[pl.ds(i * 16, 16)]
            e = tle[pl.ds(i * 16, 16)]
            valid = (lane + i * 16) < n_t
            msk = valid & (v >= blo) & (v < blo + _BKN)
            mi = msk.astype(i32)
            p = _prefix16(mi, lane)
            pos = nbv + p - mi
            posw = jnp.minimum(jnp.where(msk, pos, _BKCAP - 1), _BKCAP - 1)
            plsc.store_scatter(eb, [posw], e)
            plsc.store_scatter(lb, [posw], v - blo)
            return nbv + _lperm(p, fifteen)

        nbv = lax.fori_loop(0, (n_t + 15) // 16, bcomp, jnp.zeros((16,), i32))
        n_b = jnp.minimum(nbv[0], _BKCAP - 16)

        def ebatch(bi, c):
            pltpu.async_copy(m4_hbm.at[eb.at[pl.ds(bi * _RB, _RB)]], rows, sem).wait()

            def erow(r, c2):
                pidx = jnp.full((16,), bi * _RB + r, i32)
                l = plsc.load_gather(lb, [pidx])[0]
                ci = k * _BKN + l
                cnt2[ci, :] = cnt2[ci, :] + lane0f
                for j in range(PF // 16):
                    sl = pl.ds(j * 16, 16)
                    mv = rows[r, sl]
                    acc_s[l, sl] = acc_s[l, sl] + mv
                    acc_q[l, sl] = acc_q[l, sl] + mv * mv
                    acc_n[l, sl] = jnp.minimum(acc_n[l, sl], mv)
                    acc_x[l, sl] = jnp.maximum(acc_x[l, sl], mv)
                return c2

            lax.fori_loop(0, jnp.minimum(_RB, n_b - bi * _RB), erow, 0)
            return c

        lax.fori_loop(0, (n_b + _RB - 1) // _RB, ebatch, 0)

        pltpu.sync_copy(acc_s, s_hbm.at[pl.ds(blo, _BKN)])
        pltpu.sync_copy(acc_q, q_hbm.at[pl.ds(blo, _BKN)])
        pltpu.sync_copy(acc_n, n_hbm.at[pl.ds(blo, _BKN)])
        pltpu.sync_copy(acc_x, x_hbm.at[pl.ds(blo, _BKN)])
        return carry

    lax.fori_loop(0, _NBK, bucket, 0)

    def cgath(g, c):
        ridx = lane + g * 16
        zidx = jnp.zeros((16,), i32)
        cntv[pl.ds(g * 16, 16)] = plsc.load_gather(cnt2, [ridx, zidx])
        return c

    lax.fori_loop(0, _NPW // 16, cgath, 0)
    pltpu.sync_copy(cntv, c_hbm.at[pl.ds(lo, _NPW)])


def _aggregate(m4, dst):
    mesh = plsc.VectorSubcoreMesh(core_axis_name="c", subcore_axis_name="s")
    f = functools.partial(
        pl.kernel,
        out_type=[jax.ShapeDtypeStruct((N_NODES, PF), jnp.float32)] * 4
        + [jax.ShapeDtypeStruct((N_NODES,), jnp.float32)],
        mesh=mesh,
        compiler_params=pltpu.CompilerParams(needs_layout_passes=False),
        scratch_types=[
            pltpu.VMEM((N_EDGES,), jnp.int32),
            pltpu.VMEM((_TLCAP,), jnp.int32),
            pltpu.VMEM((_TLCAP,), jnp.int32),
            pltpu.VMEM((_BKCAP,), jnp.int32),
            pltpu.VMEM((_BKCAP + 16,), jnp.int32),
            pltpu.VMEM((_BKN, PF), jnp.float32),
            pltpu.VMEM((_BKN, PF), jnp.float32),
            pltpu.VMEM((_BKN, PF), jnp.float32),
            pltpu.VMEM((_BKN, PF), jnp.float32),
            pltpu.VMEM((_RB, PF), jnp.float32),
            pltpu.VMEM((_NPW,), jnp.float32),
            pltpu.VMEM((_NPW, 16), jnp.float32),
            pltpu.SemaphoreType.DMA,
        ],
    )(_agg_sc_body)
    s, sq, mn, mx, cnt = f(m4, dst)
    return s, sq, mn, mx, cnt


def kernel(x, edge_index, edge_attr, batch, params):
    f32 = jnp.float32
    xp = _pad2(x, N_NODES, PF).astype(jnp.bfloat16)
    p = params
    w1 = _pad2(p["mlp1"]["w"], PF, PF)
    b1 = _padb(p["mlp1"]["b"], PF)
    pre0w = p["pre"][0]["w"]
    wd = _pad2(pre0w[:F], PF, PF)
    bd = _padb(p["pre"][0]["b"], PF)
    ws_ = _pad2(pre0w[F:2 * F], PF, PF)
    emb = _pad2(p["edge_emb"], 32, 128)
    wenc = _pad2(p["edge_enc"]["w"], 128, PF)
    benc = _padb(p["edge_enc"]["b"], PF)
    wc = _pad2(pre0w[2 * F:], PF, PF)
    prew = [_pad2(p["pre"][i]["w"], PF, PF) for i in range(1, 5)]
    preb = [_padb(p["pre"][i]["b"], PF) for i in range(1, 5)]
    post0w = p["post"][0]["w"]
    wslices = [_pad2(post0w[i * F:(i + 1) * F], PF, PF) for i in range(6)]
    b0 = _padb(p["post"][0]["b"], PF)
    postw = [_pad2(p["post"][i]["w"], PF, PF) for i in range(1, 5)]
    postb = [_padb(p["post"][i]["b"], PF) for i in range(1, 5)]
    wl = _pad2(p["lin"]["w"], PF, PF)
    bl = _padb(p["lin"]["b"], PF)
    gamma = _padb(p["bn_gamma"], PF)
    beta = _padb(p["bn_beta"], PF)
    m2w1 = _pad2(p["mlp2"][0]["w"], PF, 640)
    m2b1 = _padb(p["mlp2"][0]["b"], 640)
    m2w2 = _pad2(p["mlp2"][1]["w"], 640, 128)
    m2b2 = _padb(p["mlp2"][1]["b"], 128)
    m2w3 = _pad2(p["mlp2"][2]["w"], 128, 128)
    m2b3 = _padb(p["mlp2"][2]["b"], 128)
    m3w1 = _pad2(p["mlp3"][0]["w"], PF, 640)
    m3b1 = _padb(p["mlp3"][0]["b"], 640)
    m3w2 = _pad2(p["mlp3"][1]["w"], 640, 128)
    m3b2 = _padb(p["mlp3"][1]["b"], 128)
    m3w3 = _pad2(p["mlp3"][2]["w"], 128, 128)
    m3b3 = _padb(p["mlp3"][2]["b"], 128)
    aw5 = jax.nn.softmax(p["agg_w"])
    aw = jnp.zeros((8, 128), f32).at[0, :5].set(aw5)

    dst = edge_index[1]
    src = edge_index[0]
    attr3 = edge_attr.astype(jnp.int32).reshape(N_EDGES // 768, 1, 768)
    batch3 = batch.astype(jnp.int32).reshape(N_NODES // 512, 1, 512)

    bf = jnp.bfloat16
    w1 = w1.astype(bf)
    wd = wd.astype(bf)
    ws_ = ws_.astype(bf)
    prew = [w.astype(bf) for w in prew]
    wslices = [w.astype(bf) for w in wslices]
    postw = [w.astype(bf) for w in postw]
    wl = wl.astype(bf)
    m3w1 = m3w1.astype(bf)
    m3w2 = m3w2.astype(bf)
    m3w3 = m3w3.astype(bf)
    ctab = _ctab(emb, wenc, benc, wc).astype(bf)
    h0, hd, hs = _node_pre(xp, w1, b1, wd, bd, ws_)
    m0 = _edge_gather(hd, hs, dst, src)
    m4 = _pre_chain(m0, attr3, ctab, prew, preb)
    s, sq, mn, mx, cnt = _aggregate(m4, dst)
    cnt128 = jnp.broadcast_to(cnt[:, None], (N_NODES, 128))
    o0 = _post0(h0, s, sq, mn, mx, cnt128, aw, wslices, b0)
    out, part = _post_chain(o0, postw, postb, wl, bl)
    xf_pad, pool = _finale(out, part, batch3, gamma, beta, m3w1, m3b1, m3w2, m3b2, m3w3, m3b3)
    xe_pad = _mlp2(pool, m2w1, m2b1, m2w2, m2b2, m2w3, m2b3)
    return xf_pad[:, :3], xe_pad[:, :1]


# speedup vs baseline: 1.3666x; 1.0642x over previous
import functools
import jax
import jax.numpy as jnp
from jax import lax
from jax.experimental import pallas as pl
from jax.experimental.pallas import tpu as pltpu
from jax.experimental.pallas import tpu_sc as plsc

N_NODES = 5120
N_EDGES = 15360
N_GRAPHS = 64
F = 1262
PF = 1280


def _pad2(a, r, c):
    return jnp.zeros((r, c), a.dtype).at[: a.shape[0], : a.shape[1]].set(a)


def _padb(b, c):
    z = jnp.zeros((8, c), b.dtype)
    return z.at[0, : b.shape[0]].set(b)


def _ctab_body(emb_ref, wenc_ref, benc_ref, wc_ref, out_ref):
    t = jnp.dot(emb_ref[...], wenc_ref[...], preferred_element_type=jnp.float32)
    t = t + benc_ref[0:1, :]
    out_ref[...] = jnp.dot(t, wc_ref[...], preferred_element_type=jnp.float32)


def _ctab(emb, wenc, benc, wc):
    return pl.pallas_call(
        _ctab_body,
        out_shape=jax.ShapeDtypeStruct((32, PF), jnp.float32),
    )(emb, wenc, benc, wc)


def _node_pre_body(x_ref, w1_ref, b1_ref, wd_ref, bd_ref, ws_ref, h0_ref, hd_ref, hs_ref):
    h0 = jnp.dot(x_ref[...], w1_ref[...], preferred_element_type=jnp.float32)
    h0 = jnp.maximum(h0 + b1_ref[0:1, :], 0.0)
    h0_ref[...] = h0
    hb = h0.astype(jnp.bfloat16)
    hd_ref[...] = jnp.dot(hb, wd_ref[...], preferred_element_type=jnp.float32) + bd_ref[0:1, :]
    hs_ref[...] = jnp.dot(hb, ws_ref[...], preferred_element_type=jnp.float32)


def _node_pre(x, w1, b1, wd, bd, ws):
    bm = 512
    grid = (N_NODES // bm,)
    blk = pl.BlockSpec((bm, PF), lambda i: (i, 0))
    wspec = pl.BlockSpec((PF, PF), lambda i: (0, 0))
    bspec = pl.BlockSpec((8, PF), lambda i: (0, 0))
    return pl.pallas_call(
        _node_pre_body,
        grid=grid,
        in_specs=[blk, wspec, bspec, wspec, bspec, wspec],
        out_specs=[blk, blk, blk],
        out_shape=[jax.ShapeDtypeStruct((N_NODES, PF), jnp.float32)] * 3,
    )(x, w1, b1, wd, bd, ws)


def _pre_chain_body(m0_ref, attr_ref, ctab_ref, w1, b1, w2, b2, w3, b3, w4, b4, out_ref):
    a = attr_ref[0, 0, :]
    oh = (a[:, None] == lax.broadcasted_iota(jnp.int32, (a.shape[0], 32), 1)).astype(jnp.bfloat16)
    m = m0_ref[...] + jnp.dot(oh, ctab_ref[...], preferred_element_type=jnp.float32)
    for w_ref, b_ref in ((w1, b1), (w2, b2), (w3, b3), (w4, b4)):
        m = jnp.maximum(m, 0.0).astype(jnp.bfloat16)
        m = jnp.dot(m, w_ref[...], preferred_element_type=jnp.float32) + b_ref[0:1, :]
    out_ref[...] = m


def _pre_chain(m0, attr3, ctab, ws, bs):
    bm = 768
    grid = (N_EDGES // bm,)
    blk = pl.BlockSpec((bm, PF), lambda i: (i, 0))
    aspec = pl.BlockSpec((1, 1, bm), lambda i: (i, 0, 0))
    cspec = pl.BlockSpec((32, PF), lambda i: (0, 0))
    wspec = pl.BlockSpec((PF, PF), lambda i: (0, 0))
    bspec = pl.BlockSpec((8, PF), lambda i: (0, 0))
    in_specs = [blk, aspec, cspec]
    args = [m0, attr3, ctab]
    for w, b in zip(ws, bs):
        in_specs += [wspec, bspec]
        args += [w, b]
    return pl.pallas_call(
        _pre_chain_body,
        grid=grid,
        in_specs=in_specs,
        out_specs=blk,
        out_shape=jax.ShapeDtypeStruct((N_EDGES, PF), jnp.float32),
    )(*args)


def _post0_body(h0_ref, s_ref, sq_ref, mn_ref, mx_ref, cnt_ref, aw_ref,
                wh, wsum, wmean, wmin, wmax, wstd, b_ref, out_ref):
    cnt = cnt_ref[...][:, 0:1]
    cnt_safe = jnp.maximum(cnt, 1.0)
    has = (cnt > 0.0).astype(jnp.float32)
    s = s_ref[...]
    mean = s / cnt_safe
    mn = mn_ref[...] * has
    mx = mx_ref[...] * has
    msq = sq_ref[...] / cnt_safe
    var = jnp.maximum(msq - mean * mean, 0.0)
    std = jnp.sqrt(var + 1e-5)
    a0 = aw_ref[0, 0]
    a1 = aw_ref[0, 1]
    a2 = aw_ref[0, 2]
    a3 = aw_ref[0, 3]
    a4 = aw_ref[0, 4]
    bf = jnp.bfloat16
    acc = jnp.dot(h0_ref[...].astype(bf), wh[...], preferred_element_type=jnp.float32)
    acc += jnp.dot((a0 * s).astype(bf), wsum[...], preferred_element_type=jnp.float32)
    acc += jnp.dot((a1 * mean).astype(bf), wmean[...], preferred_element_type=jnp.float32)
    acc += jnp.dot((a2 * mn).astype(bf), wmin[...], preferred_element_type=jnp.float32)
    acc += jnp.dot((a3 * mx).astype(bf), wmax[...], preferred_element_type=jnp.float32)
    acc += jnp.dot((a4 * std).astype(bf), wstd[...], preferred_element_type=jnp.float32)
    out_ref[...] = acc + b_ref[0:1, :]


def _post0(h0, s, sq, mn, mx, cnt128, aw, wslices, b0):
    bm = 256
    grid = (N_NODES // bm,)
    blk = pl.BlockSpec((bm, PF), lambda i: (i, 0))
    cspec = pl.BlockSpec((bm, 128), lambda i: (i, 0))
    awspec = pl.BlockSpec((8, 128), lambda i: (0, 0))
    wspec = pl.BlockSpec((PF, PF), lambda i: (0, 0))
    bspec = pl.BlockSpec((8, PF), lambda i: (0, 0))
    return pl.pallas_call(
        _post0_body,
        grid=grid,
        in_specs=[blk, blk, blk, blk, blk, cspec, awspec] + [wspec] * 6 + [bspec],
        out_specs=blk,
        out_shape=jax.ShapeDtypeStruct((N_NODES, PF), jnp.float32),
    )(h0, s, sq, mn, mx, cnt128, aw, *wslices, b0)


def _post_chain_body(x_ref, w1, b1, w2, b2, w3, b3, w4, b4, wl, bl, out_ref, part_ref):
    m = x_ref[...]
    for w_ref, b_ref in ((w1, b1), (w2, b2), (w3, b3), (w4, b4)):
        m = jnp.maximum(m, 0.0).astype(jnp.bfloat16)
        m = jnp.dot(m, w_ref[...], preferred_element_type=jnp.float32) + b_ref[0:1, :]
    m = jnp.dot(m.astype(jnp.bfloat16), wl[...], preferred_element_type=jnp.float32) + bl[0:1, :]
    out_ref[...] = m
    part_ref[0, 0, :] = jnp.sum(m, axis=0)
    part_ref[0, 1, :] = jnp.sum(m * m, axis=0)


def _post_chain(x, ws, bs, wl, bl):
    bm = 512
    nb = N_NODES // bm
    grid = (nb,)
    blk = pl.BlockSpec((bm, PF), lambda i: (i, 0))
    wspec = pl.BlockSpec((PF, PF), lambda i: (0, 0))
    bspec = pl.BlockSpec((8, PF), lambda i: (0, 0))
    in_specs = [blk]
    args = [x]
    for w, b in zip(ws, bs):
        in_specs += [wspec, bspec]
        args += [w, b]
    in_specs += [wspec, bspec]
    args += [wl, bl]
    return pl.pallas_call(
        _post_chain_body,
        grid=grid,
        in_specs=in_specs,
        out_specs=[blk, pl.BlockSpec((1, 2, PF), lambda i: (i, 0, 0))],
        out_shape=[jax.ShapeDtypeStruct((N_NODES, PF), jnp.float32),
                   jax.ShapeDtypeStruct((nb, 2, PF), jnp.float32)],
    )(*args)


def _finale_body(x_ref, part_ref, batch_ref, g_ref, be_ref, w1, b1, w2, b2, w3, b3,
                 xf_ref, pool_ref):
    i = pl.program_id(0)
    colsum = jnp.sum(part_ref[:, 0, :], axis=0, keepdims=True)
    colsq = jnp.sum(part_ref[:, 1, :], axis=0, keepdims=True)
    mu = colsum / float(N_NODES)
    var = colsq / float(N_NODES) - mu * mu
    rstd = lax.rsqrt(var + 1e-5)
    h = (x_ref[...] - mu) * rstd * g_ref[0:1, :] + be_ref[0:1, :]
    h = jnp.maximum(h, 0.0)
    b = batch_ref[0, 0, :]
    b2d = jnp.broadcast_to(b[None, :], (N_GRAPHS, b.shape[0]))
    g2d = lax.broadcasted_iota(jnp.int32, (N_GRAPHS, b.shape[0]), 0)
    oh = (b2d == g2d).astype(jnp.float32)

    @pl.when(i == 0)
    def _():
        pool_ref[...] = jnp.zeros_like(pool_ref)

    pool_ref[...] += jnp.dot(oh, h, preferred_element_type=jnp.float32)
    f = jnp.maximum(jnp.dot(h.astype(jnp.bfloat16), w1[...], preferred_element_type=jnp.float32) + b1[0:1, :], 0.0)
    f = jnp.maximum(jnp.dot(f.astype(jnp.bfloat16), w2[...], preferred_element_type=jnp.float32) + b2[0:1, :], 0.0)
    xf_ref[...] = jnp.dot(f.astype(jnp.bfloat16), w3[...], preferred_element_type=jnp.float32) + b3[0:1, :]


def _finale(x, part, batch3, gamma, beta, w1, b1, w2, b2, w3, b3):
    bm = 512
    nb = N_NODES // bm
    grid = (nb,)
    blk = pl.BlockSpec((bm, PF), lambda i: (i, 0))
    return pl.pallas_call(
        _finale_body,
        grid=grid,
        in_specs=[blk,
                  pl.BlockSpec((nb, 2, PF), lambda i: (0, 0, 0)),
                  pl.BlockSpec((1, 1, bm), lambda i: (i, 0, 0)),
                  pl.BlockSpec((8, PF), lambda i: (0, 0)),
                  pl.BlockSpec((8, PF), lambda i: (0, 0)),
                  pl.BlockSpec((PF, 640), lambda i: (0, 0)),
                  pl.BlockSpec((8, 640), lambda i: (0, 0)),
                  pl.BlockSpec((640, 128), lambda i: (0, 0)),
                  pl.BlockSpec((8, 128), lambda i: (0, 0)),
                  pl.BlockSpec((128, 128), lambda i: (0, 0)),
                  pl.BlockSpec((8, 128), lambda i: (0, 0))],
        out_specs=[pl.BlockSpec((bm, 128), lambda i: (i, 0)),
                   pl.BlockSpec((N_GRAPHS, PF), lambda i: (0, 0))],
        out_shape=[jax.ShapeDtypeStruct((N_NODES, 128), jnp.float32),
                   jax.ShapeDtypeStruct((N_GRAPHS, PF), jnp.float32)],
    )(x, part, batch3, gamma, beta, w1, b1, w2, b2, w3, b3)


def _mlp2_body(p_ref, w1, b1, w2, b2, w3, b3, out_ref):
    t = jnp.maximum(jnp.dot(p_ref[...], w1[...], preferred_element_type=jnp.float32) + b1[0:1, :], 0.0)
    t = jnp.maximum(jnp.dot(t, w2[...], preferred_element_type=jnp.float32) + b2[0:1, :], 0.0)
    out_ref[...] = jnp.dot(t, w3[...], preferred_element_type=jnp.float32) + b3[0:1, :]


def _mlp2(pool, w1, b1, w2, b2, w3, b3):
    return pl.pallas_call(
        _mlp2_body,
        out_shape=jax.ShapeDtypeStruct((N_GRAPHS, 128), jnp.float32),
    )(pool, w1, b1, w2, b2, w3, b3)


_NW = 32
_EPW = N_EDGES // _NW
_GB = 24


def _gather_sc_body(hd_hbm, hs_hbm, dst_hbm, src_hbm, out_hbm,
                    dsti, srci, bufa0, bufa1, bufb0, bufb1,
                    sa0, sa1, sb0, sb1, so0, so1):
    wid = lax.axis_index("s") * 2 + lax.axis_index("c")
    base = wid * _EPW
    pltpu.sync_copy(dst_hbm.at[pl.ds(base, _EPW)], dsti)
    pltpu.sync_copy(src_hbm.at[pl.ds(base, _EPW)], srci)
    bufa = [bufa0, bufa1]
    bufb = [bufb0, bufb1]
    sa = [sa0, sa1]
    sb = [sb0, sb1]
    so = [so0, so1]
    nb = _EPW // _GB
    outh = [None, None]

    def issue(b):
        sl = b % 2
        if outh[sl] is not None:
            outh[sl].wait()
            outh[sl] = None
        ca = pltpu.async_copy(hd_hbm.at[dsti.at[pl.ds(b * _GB, _GB)]], bufa[sl], sa[sl])
        cb = pltpu.async_copy(hs_hbm.at[srci.at[pl.ds(b * _GB, _GB)]], bufb[sl], sb[sl])
        return ca, cb

    pend = issue(0)
    for b in range(nb):
        sl = b % 2
        nxt = issue(b + 1) if b + 1 < nb else None
        pend[0].wait()
        pend[1].wait()
        ba = bufa[sl]
        bb = bufb[sl]

        def row_body(r, c2, ba=ba, bb=bb):
            for j in range(PF // 16):
                slc = pl.ds(j * 16, 16)
                ba[r, slc] = ba[r, slc] + bb[r, slc]
            return c2

        lax.fori_loop(0, _GB, row_body, 0)
        outh[sl] = pltpu.async_copy(ba, out_hbm.at[pl.ds(base + b * _GB, _GB)], so[sl])
        pend = nxt
    if outh[0] is not None:
        outh[0].wait()
    if outh[1] is not None:
        outh[1].wait()


def _edge_gather(hd, hs, dst, src):
    mesh = plsc.VectorSubcoreMesh(core_axis_name="c", subcore_axis_name="s")
    f = functools.partial(
        pl.kernel,
        out_type=jax.ShapeDtypeStruct((N_EDGES, PF), jnp.float32),
        mesh=mesh,
        scratch_types=[
            pltpu.VMEM((_EPW,), jnp.int32),
            pltpu.VMEM((_EPW,), jnp.int32),
            pltpu.VMEM((_GB, PF), jnp.float32),
            pltpu.VMEM((_GB, PF), jnp.float32),
            pltpu.VMEM((_GB, PF), jnp.float32),
            pltpu.VMEM((_GB, PF), jnp.float32),
            pltpu.SemaphoreType.DMA,
            pltpu.SemaphoreType.DMA,
            pltpu.SemaphoreType.DMA,
            pltpu.SemaphoreType.DMA,
            pltpu.SemaphoreType.DMA,
            pltpu.SemaphoreType.DMA,
        ],
    )(_gather_sc_body)
    return f(hd, hs, dst, src)


_NPW = N_NODES // _NW
_BKN = 16
_NBK = _NPW // _BKN
_TLCAP = 1024
_BKCAP = 256
_RB = 16
_FINF = 3.0e38
_DCH = 1920


def _lperm(v, idx):
    return lax.gather(
        v, idx[:, None],
        dimension_numbers=lax.GatherDimensionNumbers(
            offset_dims=(), collapsed_slice_dims=(0,), start_index_map=(0,)),
        slice_sizes=(1,), mode=lax.GatherScatterMode.PROMISE_IN_BOUNDS)


def _prefix16(mi, lane):
    p = mi
    for k in (1, 2, 4, 8):
        sh = _lperm(p, jnp.maximum(lane - k, 0))
        p = p + jnp.where(lane >= k, sh, 0)
    return p


def _agg_sc_body(m4_hbm, dst_hbm, s_hbm, q_hbm, n_hbm, x_hbm, c_hbm,
                 dstv, tle, tld, eb, lb, acc_s, acc_q, acc_n, acc_x,
                 rows, cntv, cnt2, sem):
    i32 = jnp.int32
    f32 = jnp.float32
    wid = lax.axis_index("s") * 2 + lax.axis_index("c")
    lo = wid * _NPW
    lane = lax.iota(i32, 16)
    lane0f = (lane == 0).astype(f32)
    fifteen = jnp.full((16,), 15, i32)

    def czero(i, c):
        cnt2[i, :] = jnp.zeros((16,), f32)
        return c

    lax.fori_loop(0, _NPW, czero, 0)

    def wchunk(ch, nvec0):
        pltpu.sync_copy(dst_hbm.at[pl.ds(ch * _DCH, _DCH)], dstv)

        def wcomp(i, nvec):
            v = dstv[pl.ds(i * 16, 16)]
            inb = (v >= lo) & (v < lo + _NPW)
            mi = inb.astype(i32)
            p = _prefix16(mi, lane)
            pos = nvec + p - mi
            posw = jnp.minimum(jnp.where(inb, pos, _TLCAP - 1), _TLCAP - 1)
            plsc.store_scatter(tle, [posw], lane + ch * _DCH + i * 16)
            plsc.store_scatter(tld, [posw], v)
            return nvec + _lperm(p, fifteen)

        return lax.fori_loop(0, _DCH // 16, wcomp, nvec0)

    nvec = lax.fori_loop(0, N_EDGES // _DCH, wchunk, jnp.zeros((16,), i32))
    n_t = jnp.minimum(nvec[0], _TLCAP - 1)

    def bucket(k, carry):
        blo = lo + k * _BKN

        def initrow(r, c):
            for j in range(PF // 16):
                sl = pl.ds(j * 16, 16)
                acc_s[r, sl] = jnp.zeros((16,), f32)
                acc_q[r, sl] = jnp.zeros((16,), f32)
                acc_n[r, sl] = jnp.full((16,), _FINF, f32)
                acc_x[r, sl] = jnp.full((16,), -_FINF, f32)
            return c

        lax.fori_loop(0, _BKN, initrow, 0)

        def ezero(i, c):
            eb[pl.ds(i * 16, 16)] = jnp.zeros((16,), i32)
            return c

        lax.fori_loop(0, _BKCAP // 16, ezero, 0)

        def bcomp(i, nbv):
            v = tld[pl.ds(i * 16, 16)]
            e = tle[pl.ds(i * 16, 16)]
            valid = (lane + i * 16) < n_t
            msk = valid & (v >= blo) & (v < blo + _BKN)
            mi = msk.astype(i32)
            p = _prefix16(mi, lane)
            pos = nbv + p - mi
            posw = jnp.minimum(jnp.where(msk, pos, _BKCAP - 1), _BKCAP - 1)
            plsc.store_scatter(eb, [posw], e)
            plsc.store_scatter(lb, [posw], v - blo)
            return nbv + _lperm(p, fifteen)

        nbv = lax.fori_loop(0, (n_t + 15) // 16, bcomp, jnp.zeros((16,), i32))
        n_b = jnp.minimum(nbv[0], _BKCAP - 16)

        def ebatch(bi, c):
            pltpu.async_copy(m4_hbm.at[eb.at[pl.ds(bi * _RB, _RB)]], rows, sem).wait()

            def erow(r, c2):
                pidx = jnp.full((16,), bi * _RB + r, i32)
                l = plsc.load_gather(lb, [pidx])[0]
                ci = k * _BKN + l
                cnt2[ci, :] = cnt2[ci, :] + lane0f
                for j in range(PF // 16):
                    sl = pl.ds(j * 16, 16)
                    mv = rows[r, sl]
                    acc_s[l, sl] = acc_s[l, sl] + mv
                    acc_q[l, sl] = acc_q[l, sl] + mv * mv
                    acc_n[l, sl] = jnp.minimum(acc_n[l, sl], mv)
                    acc_x[l, sl] = jnp.maximum(acc_x[l, sl], mv)
                return c2

            lax.fori_loop(0, jnp.minimum(_RB, n_b - bi * _RB), erow, 0)
            return c

        lax.fori_loop(0, (n_b + _RB - 1) // _RB, ebatch, 0)

        pltpu.sync_copy(acc_s, s_hbm.at[pl.ds(blo, _BKN)])
        pltpu.sync_copy(acc_q, q_hbm.at[pl.ds(blo, _BKN)])
        pltpu.sync_copy(acc_n, n_hbm.at[pl.ds(blo, _BKN)])
        pltpu.sync_copy(acc_x, x_hbm.at[pl.ds(blo, _BKN)])
        return carry

    lax.fori_loop(0, _NBK, bucket, 0)

    def cgath(g, c):
        ridx = lane + g * 16
        zidx = jnp.zeros((16,), i32)
        cntv[pl.ds(g * 16, 16)] = plsc.load_gather(cnt2, [ridx, zidx])
        return c

    lax.fori_loop(0, _NPW // 16, cgath, 0)
    pltpu.sync_copy(cntv, c_hbm.at[pl.ds(lo, _NPW)])


def _aggregate(m4, dst):
    mesh = plsc.VectorSubcoreMesh(core_axis_name="c", subcore_axis_name="s")
    f = functools.partial(
        pl.kernel,
        out_type=[jax.ShapeDtypeStruct((N_NODES, PF), jnp.float32)] * 4
        + [jax.ShapeDtypeStruct((N_NODES,), jnp.float32)],
        mesh=mesh,
        compiler_params=pltpu.CompilerParams(needs_layout_passes=False),
        scratch_types=[
            pltpu.VMEM((_DCH,), jnp.int32),
            pltpu.VMEM((_TLCAP,), jnp.int32),
            pltpu.VMEM((_TLCAP,), jnp.int32),
            pltpu.VMEM((_BKCAP,), jnp.int32),
            pltpu.VMEM((_BKCAP + 16,), jnp.int32),
            pltpu.VMEM((_BKN, PF), jnp.float32),
            pltpu.VMEM((_BKN, PF), jnp.float32),
            pltpu.VMEM((_BKN, PF), jnp.float32),
            pltpu.VMEM((_BKN, PF), jnp.float32),
            pltpu.VMEM((_RB, PF), jnp.float32),
            pltpu.VMEM((_NPW,), jnp.float32),
            pltpu.VMEM((_NPW, 16), jnp.float32),
            pltpu.SemaphoreType.DMA,
        ],
    )(_agg_sc_body)
    s, sq, mn, mx, cnt = f(m4, dst)
    return s, sq, mn, mx, cnt


def kernel(x, edge_index, edge_attr, batch, params):
    f32 = jnp.float32
    xp = _pad2(x, N_NODES, PF).astype(jnp.bfloat16)
    p = params
    w1 = _pad2(p["mlp1"]["w"], PF, PF)
    b1 = _padb(p["mlp1"]["b"], PF)
    pre0w = p["pre"][0]["w"]
    wd = _pad2(pre0w[:F], PF, PF)
    bd = _padb(p["pre"][0]["b"], PF)
    ws_ = _pad2(pre0w[F:2 * F], PF, PF)
    emb = _pad2(p["edge_emb"], 32, 128)
    wenc = _pad2(p["edge_enc"]["w"], 128, PF)
    benc = _padb(p["edge_enc"]["b"], PF)
    wc = _pad2(pre0w[2 * F:], PF, PF)
    prew = [_pad2(p["pre"][i]["w"], PF, PF) for i in range(1, 5)]
    preb = [_padb(p["pre"][i]["b"], PF) for i in range(1, 5)]
    post0w = p["post"][0]["w"]
    wslices = [_pad2(post0w[i * F:(i + 1) * F], PF, PF) for i in range(6)]
    b0 = _padb(p["post"][0]["b"], PF)
    postw = [_pad2(p["post"][i]["w"], PF, PF) for i in range(1, 5)]
    postb = [_padb(p["post"][i]["b"], PF) for i in range(1, 5)]
    wl = _pad2(p["lin"]["w"], PF, PF)
    bl = _padb(p["lin"]["b"], PF)
    gamma = _padb(p["bn_gamma"], PF)
    beta = _padb(p["bn_beta"], PF)
    m2w1 = _pad2(p["mlp2"][0]["w"], PF, 640)
    m2b1 = _padb(p["mlp2"][0]["b"], 640)
    m2w2 = _pad2(p["mlp2"][1]["w"], 640, 128)
    m2b2 = _padb(p["mlp2"][1]["b"], 128)
    m2w3 = _pad2(p["mlp2"][2]["w"], 128, 128)
    m2b3 = _padb(p["mlp2"][2]["b"], 128)
    m3w1 = _pad2(p["mlp3"][0]["w"], PF, 640)
    m3b1 = _padb(p["mlp3"][0]["b"], 640)
    m3w2 = _pad2(p["mlp3"][1]["w"], 640, 128)
    m3b2 = _padb(p["mlp3"][1]["b"], 128)
    m3w3 = _pad2(p["mlp3"][2]["w"], 128, 128)
    m3b3 = _padb(p["mlp3"][2]["b"], 128)
    aw5 = jax.nn.softmax(p["agg_w"])
    aw = jnp.zeros((8, 128), f32).at[0, :5].set(aw5)

    dst = edge_index[1]
    src = edge_index[0]
    attr3 = edge_attr.astype(jnp.int32).reshape(N_EDGES // 768, 1, 768)
    batch3 = batch.astype(jnp.int32).reshape(N_NODES // 512, 1, 512)

    bf = jnp.bfloat16
    w1 = w1.astype(bf)
    wd = wd.astype(bf)
    ws_ = ws_.astype(bf)
    prew = [w.astype(bf) for w in prew]
    wslices = [w.astype(bf) for w in wslices]
    postw = [w.astype(bf) for w in postw]
    wl = wl.astype(bf)
    m3w1 = m3w1.astype(bf)
    m3w2 = m3w2.astype(bf)
    m3w3 = m3w3.astype(bf)
    ctab = _ctab(emb, wenc, benc, wc).astype(bf)
    h0, hd, hs = _node_pre(xp, w1, b1, wd, bd, ws_)
    m0 = _edge_gather(hd, hs, dst, src)
    m4 = _pre_chain(m0, attr3, ctab, prew, preb)
    s, sq, mn, mx, cnt = _aggregate(m4, dst)
    cnt128 = jnp.broadcast_to(cnt[:, None], (N_NODES, 128))
    o0 = _post0(h0, s, sq, mn, mx, cnt128, aw, wslices, b0)
    out, part = _post_chain(o0, postw, postb, wl, bl)
    xf_pad, pool = _finale(out, part, batch3, gamma, beta, m3w1, m3b1, m3w2, m3b2, m3w3, m3b3)
    xe_pad = _mlp2(pool, m2w1, m2b1, m2w2, m2b2, m2w3, m2b3)
    return xf_pad[:, :3], xe_pad[:, :1]


# speedup vs baseline: 1.4094x; 1.0313x over previous
import functools
import jax
import jax.numpy as jnp
from jax import lax
from jax.experimental import pallas as pl
from jax.experimental.pallas import tpu as pltpu
from jax.experimental.pallas import tpu_sc as plsc

N_NODES = 5120
N_EDGES = 15360
N_GRAPHS = 64
F = 1262
PF = 1280


def _pad2(a, r, c):
    return jnp.zeros((r, c), a.dtype).at[: a.shape[0], : a.shape[1]].set(a)


def _padb(b, c):
    z = jnp.zeros((8, c), b.dtype)
    return z.at[0, : b.shape[0]].set(b)


def _ctab_body(emb_ref, wenc_ref, benc_ref, wc_ref, out_ref):
    t = jnp.dot(emb_ref[...], wenc_ref[...], preferred_element_type=jnp.float32)
    t = t + benc_ref[0:1, :]
    out_ref[...] = jnp.dot(t, wc_ref[...], preferred_element_type=jnp.float32)


def _ctab(emb, wenc, benc, wc):
    return pl.pallas_call(
        _ctab_body,
        out_shape=jax.ShapeDtypeStruct((32, PF), jnp.float32),
    )(emb, wenc, benc, wc)


def _node_pre_body(x_ref, w1_ref, b1_ref, wd_ref, bd_ref, ws_ref, h0_ref, hd_ref, hs_ref):
    h0 = jnp.dot(x_ref[...], w1_ref[...], preferred_element_type=jnp.float32)
    h0 = jnp.maximum(h0 + b1_ref[0:1, :], 0.0)
    h0_ref[...] = h0
    hb = h0.astype(jnp.bfloat16)
    hd_ref[...] = jnp.dot(hb, wd_ref[...], preferred_element_type=jnp.float32) + bd_ref[0:1, :]
    hs_ref[...] = jnp.dot(hb, ws_ref[...], preferred_element_type=jnp.float32)


def _node_pre(x, w1, b1, wd, bd, ws):
    bm = 512
    grid = (N_NODES // bm,)
    blk = pl.BlockSpec((bm, PF), lambda i: (i, 0))
    wspec = pl.BlockSpec((PF, PF), lambda i: (0, 0))
    bspec = pl.BlockSpec((8, PF), lambda i: (0, 0))
    return pl.pallas_call(
        _node_pre_body,
        grid=grid,
        in_specs=[blk, wspec, bspec, wspec, bspec, wspec],
        out_specs=[blk, blk, blk],
        out_shape=[jax.ShapeDtypeStruct((N_NODES, PF), jnp.float32)] * 3,
    )(x, w1, b1, wd, bd, ws)


def _pre_chain_body(m0_ref, attr_ref, ctab_ref, w1, b1, w2, b2, w3, b3, w4, b4, out_ref):
    a = attr_ref[0, 0, :]
    oh = (a[:, None] == lax.broadcasted_iota(jnp.int32, (a.shape[0], 32), 1)).astype(jnp.bfloat16)
    m = m0_ref[...] + jnp.dot(oh, ctab_ref[...], preferred_element_type=jnp.float32)
    for w_ref, b_ref in ((w1, b1), (w2, b2), (w3, b3), (w4, b4)):
        m = jnp.maximum(m, 0.0).astype(jnp.bfloat16)
        m = jnp.dot(m, w_ref[...], preferred_element_type=jnp.float32) + b_ref[0:1, :]
    out_ref[...] = m


def _pre_chain(m0, attr3, ctab, ws, bs):
    bm = 768
    grid = (N_EDGES // bm,)
    blk = pl.BlockSpec((bm, PF), lambda i: (i, 0))
    aspec = pl.BlockSpec((1, 1, bm), lambda i: (i, 0, 0))
    cspec = pl.BlockSpec((32, PF), lambda i: (0, 0))
    wspec = pl.BlockSpec((PF, PF), lambda i: (0, 0))
    bspec = pl.BlockSpec((8, PF), lambda i: (0, 0))
    in_specs = [blk, aspec, cspec]
    args = [m0, attr3, ctab]
    for w, b in zip(ws, bs):
        in_specs += [wspec, bspec]
        args += [w, b]
    return pl.pallas_call(
        _pre_chain_body,
        grid=grid,
        in_specs=in_specs,
        out_specs=blk,
        out_shape=jax.ShapeDtypeStruct((N_EDGES, PF), jnp.float32),
    )(*args)


def _post0_body(h0_ref, s_ref, sq_ref, mn_ref, mx_ref, cnt_ref, aw_ref,
                wh, wsum, wmean, wmin, wmax, wstd, b_ref, out_ref):
    cnt = cnt_ref[...][:, 0:1]
    cnt_safe = jnp.maximum(cnt, 1.0)
    has = (cnt > 0.0).astype(jnp.float32)
    s = s_ref[...]
    mean = s / cnt_safe
    mn = mn_ref[...] * has
    mx = mx_ref[...] * has
    msq = sq_ref[...] / cnt_safe
    var = jnp.maximum(msq - mean * mean, 0.0)
    std = jnp.sqrt(var + 1e-5)
    a0 = aw_ref[0, 0]
    a1 = aw_ref[0, 1]
    a2 = aw_ref[0, 2]
    a3 = aw_ref[0, 3]
    a4 = aw_ref[0, 4]
    bf = jnp.bfloat16
    acc = jnp.dot(h0_ref[...].astype(bf), wh[...], preferred_element_type=jnp.float32)
    acc += jnp.dot((a0 * s).astype(bf), wsum[...], preferred_element_type=jnp.float32)
    acc += jnp.dot((a1 * mean).astype(bf), wmean[...], preferred_element_type=jnp.float32)
    acc += jnp.dot((a2 * mn).astype(bf), wmin[...], preferred_element_type=jnp.float32)
    acc += jnp.dot((a3 * mx).astype(bf), wmax[...], preferred_element_type=jnp.float32)
    acc += jnp.dot((a4 * std).astype(bf), wstd[...], preferred_element_type=jnp.float32)
    out_ref[...] = acc + b_ref[0:1, :]


def _post0(h0, s, sq, mn, mx, cnt128, aw, wslices, b0):
    bm = 256
    grid = (N_NODES // bm,)
    blk = pl.BlockSpec((bm, PF), lambda i: (i, 0))
    cspec = pl.BlockSpec((bm, 128), lambda i: (i, 0))
    awspec = pl.BlockSpec((8, 128), lambda i: (0, 0))
    wspec = pl.BlockSpec((PF, PF), lambda i: (0, 0))
    bspec = pl.BlockSpec((8, PF), lambda i: (0, 0))
    return pl.pallas_call(
        _post0_body,
        grid=grid,
        in_specs=[blk, blk, blk, blk, blk, cspec, awspec] + [wspec] * 6 + [bspec],
        out_specs=blk,
        out_shape=jax.ShapeDtypeStruct((N_NODES, PF), jnp.float32),
    )(h0, s, sq, mn, mx, cnt128, aw, *wslices, b0)


def _post_chain_body(x_ref, w1, b1, w2, b2, w3, b3, w4, b4, wl, bl, out_ref, part_ref):
    m = x_ref[...]
    for w_ref, b_ref in ((w1, b1), (w2, b2), (w3, b3), (w4, b4)):
        m = jnp.maximum(m, 0.0).astype(jnp.bfloat16)
        m = jnp.dot(m, w_ref[...], preferred_element_type=jnp.float32) + b_ref[0:1, :]
    m = jnp.dot(m.astype(jnp.bfloat16), wl[...], preferred_element_type=jnp.float32) + bl[0:1, :]
    out_ref[...] = m
    part_ref[0, 0, :] = jnp.sum(m, axis=0)
    part_ref[0, 1, :] = jnp.sum(m * m, axis=0)


def _post_chain(x, ws, bs, wl, bl):
    bm = 512
    nb = N_NODES // bm
    grid = (nb,)
    blk = pl.BlockSpec((bm, PF), lambda i: (i, 0))
    wspec = pl.BlockSpec((PF, PF), lambda i: (0, 0))
    bspec = pl.BlockSpec((8, PF), lambda i: (0, 0))
    in_specs = [blk]
    args = [x]
    for w, b in zip(ws, bs):
        in_specs += [wspec, bspec]
        args += [w, b]
    in_specs += [wspec, bspec]
    args += [wl, bl]
    return pl.pallas_call(
        _post_chain_body,
        grid=grid,
        in_specs=in_specs,
        out_specs=[blk, pl.BlockSpec((1, 2, PF), lambda i: (i, 0, 0))],
        out_shape=[jax.ShapeDtypeStruct((N_NODES, PF), jnp.float32),
                   jax.ShapeDtypeStruct((nb, 2, PF), jnp.float32)],
    )(*args)


def _finale_body(x_ref, part_ref, batch_ref, g_ref, be_ref, w1, b1, w2, b2, w3, b3,
                 xf_ref, pool_ref):
    i = pl.program_id(0)
    colsum = jnp.sum(part_ref[:, 0, :], axis=0, keepdims=True)
    colsq = jnp.sum(part_ref[:, 1, :], axis=0, keepdims=True)
    mu = colsum / float(N_NODES)
    var = colsq / float(N_NODES) - mu * mu
    rstd = lax.rsqrt(var + 1e-5)
    h = (x_ref[...] - mu) * rstd * g_ref[0:1, :] + be_ref[0:1, :]
    h = jnp.maximum(h, 0.0)
    b = batch_ref[0, 0, :]
    b2d = jnp.broadcast_to(b[None, :], (N_GRAPHS, b.shape[0]))
    g2d = lax.broadcasted_iota(jnp.int32, (N_GRAPHS, b.shape[0]), 0)
    oh = (b2d == g2d).astype(jnp.float32)

    @pl.when(i == 0)
    def _():
        pool_ref[...] = jnp.zeros_like(pool_ref)

    pool_ref[...] += jnp.dot(oh, h, preferred_element_type=jnp.float32)
    f = jnp.maximum(jnp.dot(h.astype(jnp.bfloat16), w1[...], preferred_element_type=jnp.float32) + b1[0:1, :], 0.0)
    f = jnp.maximum(jnp.dot(f.astype(jnp.bfloat16), w2[...], preferred_element_type=jnp.float32) + b2[0:1, :], 0.0)
    xf_ref[...] = jnp.dot(f.astype(jnp.bfloat16), w3[...], preferred_element_type=jnp.float32) + b3[0:1, :]


def _finale(x, part, batch3, gamma, beta, w1, b1, w2, b2, w3, b3):
    bm = 512
    nb = N_NODES // bm
    grid = (nb,)
    blk = pl.BlockSpec((bm, PF), lambda i: (i, 0))
    return pl.pallas_call(
        _finale_body,
        grid=grid,
        in_specs=[blk,
                  pl.BlockSpec((nb, 2, PF), lambda i: (0, 0, 0)),
                  pl.BlockSpec((1, 1, bm), lambda i: (i, 0, 0)),
                  pl.BlockSpec((8, PF), lambda i: (0, 0)),
                  pl.BlockSpec((8, PF), lambda i: (0, 0)),
                  pl.BlockSpec((PF, 640), lambda i: (0, 0)),
                  pl.BlockSpec((8, 640), lambda i: (0, 0)),
                  pl.BlockSpec((640, 128), lambda i: (0, 0)),
                  pl.BlockSpec((8, 128), lambda i: (0, 0)),
                  pl.BlockSpec((128, 128), lambda i: (0, 0)),
                  pl.BlockSpec((8, 128), lambda i: (0, 0))],
        out_specs=[pl.BlockSpec((bm, 128), lambda i: (i, 0)),
                   pl.BlockSpec((N_GRAPHS, PF), lambda i: (0, 0))],
        out_shape=[jax.ShapeDtypeStruct((N_NODES, 128), jnp.float32),
                   jax.ShapeDtypeStruct((N_GRAPHS, PF), jnp.float32)],
    )(x, part, batch3, gamma, beta, w1, b1, w2, b2, w3, b3)


def _mlp2_body(p_ref, w1, b1, w2, b2, w3, b3, out_ref):
    t = jnp.maximum(jnp.dot(p_ref[...], w1[...], preferred_element_type=jnp.float32) + b1[0:1, :], 0.0)
    t = jnp.maximum(jnp.dot(t, w2[...], preferred_element_type=jnp.float32) + b2[0:1, :], 0.0)
    out_ref[...] = jnp.dot(t, w3[...], preferred_element_type=jnp.float32) + b3[0:1, :]


def _mlp2(pool, w1, b1, w2, b2, w3, b3):
    return pl.pallas_call(
        _mlp2_body,
        out_shape=jax.ShapeDtypeStruct((N_GRAPHS, 128), jnp.float32),
    )(pool, w1, b1, w2, b2, w3, b3)


_NW = 32
_EPW = N_EDGES // _NW
_GB = 24


def _gather_sc_body(hd_hbm, hs_hbm, dst_hbm, src_hbm, out_hbm,
                    dsti, srci, bufa0, bufa1, bufb0, bufb1,
                    sa0, sa1, sb0, sb1, so0, so1):
    wid = lax.axis_index("s") * 2 + lax.axis_index("c")
    base = wid * _EPW
    pltpu.sync_copy(dst_hbm.at[pl.ds(base, _EPW)], dsti)
    pltpu.sync_copy(src_hbm.at[pl.ds(base, _EPW)], srci)
    bufa = [bufa0, bufa1]
    bufb = [bufb0, bufb1]
    sa = [sa0, sa1]
    sb = [sb0, sb1]
    so = [so0, so1]
    nb = _EPW // _GB
    outh = [None, None]

    def issue(b):
        sl = b % 2
        if outh[sl] is not None:
            outh[sl].wait()
            outh[sl] = None
        ca = pltpu.async_copy(hd_hbm.at[dsti.at[pl.ds(b * _GB, _GB)]], bufa[sl], sa[sl])
        cb = pltpu.async_copy(hs_hbm.at[srci.at[pl.ds(b * _GB, _GB)]], bufb[sl], sb[sl])
        return ca, cb

    pend = issue(0)
    for b in range(nb):
        sl = b % 2
        nxt = issue(b + 1) if b + 1 < nb else None
        pend[0].wait()
        pend[1].wait()
        ba = bufa[sl]
        bb = bufb[sl]

        def row_body(r, c2, ba=ba, bb=bb):
            for j in range(PF // 16):
                slc = pl.ds(j * 16, 16)
                ba[r, slc] = ba[r, slc] + bb[r, slc]
            return c2

        lax.fori_loop(0, _GB, row_body, 0)
        outh[sl] = pltpu.async_copy(ba, out_hbm.at[pl.ds(base + b * _GB, _GB)], so[sl])
        pend = nxt
    if outh[0] is not None:
        outh[0].wait()
    if outh[1] is not None:
        outh[1].wait()


def _edge_gather(hd, hs, dst, src):
    mesh = plsc.VectorSubcoreMesh(core_axis_name="c", subcore_axis_name="s")
    f = functools.partial(
        pl.kernel,
        out_type=jax.ShapeDtypeStruct((N_EDGES, PF), jnp.float32),
        mesh=mesh,
        scratch_types=[
            pltpu.VMEM((_EPW,), jnp.int32),
            pltpu.VMEM((_EPW,), jnp.int32),
            pltpu.VMEM((_GB, PF), jnp.float32),
            pltpu.VMEM((_GB, PF), jnp.float32),
            pltpu.VMEM((_GB, PF), jnp.float32),
            pltpu.VMEM((_GB, PF), jnp.float32),
            pltpu.SemaphoreType.DMA,
            pltpu.SemaphoreType.DMA,
            pltpu.SemaphoreType.DMA,
            pltpu.SemaphoreType.DMA,
            pltpu.SemaphoreType.DMA,
            pltpu.SemaphoreType.DMA,
        ],
    )(_gather_sc_body)
    return f(hd, hs, dst, src)


_NPW = N_NODES // _NW
_BKN = 16
_NBK = _NPW // _BKN
_TLCAP = 1024
_BKCAP = 256
_RB = 16
_FINF = 3.0e38
_DCH = 1920


def _lperm(v, idx):
    return lax.gather(
        v, idx[:, None],
        dimension_numbers=lax.GatherDimensionNumbers(
            offset_dims=(), collapsed_slice_dims=(0,), start_index_map=(0,)),
        slice_sizes=(1,), mode=lax.GatherScatterMode.PROMISE_IN_BOUNDS)


def _prefix16(mi, lane):
    p = mi
    for k in (1, 2, 4, 8):
        sh = _lperm(p, jnp.maximum(lane - k, 0))
        p = p + jnp.where(lane >= k, sh, 0)
    return p


def _agg_sc_body(m4_hbm, dst_hbm, s_hbm, q_hbm, n_hbm, x_hbm, c_hbm,
                 dstv, tle, tld, eb, lb, acc_s, acc_q, acc_n, acc_x,
                 rows, cntv, cnt2, sem):
    i32 = jnp.int32
    f32 = jnp.float32
    wid = lax.axis_index("s") * 2 + lax.axis_index("c")
    lo = wid * _NPW
    lane = lax.iota(i32, 16)
    lane0f = (lane == 0).astype(f32)
    fifteen = jnp.full((16,), 15, i32)

    def czero(i, c):
        cnt2[i, :] = jnp.zeros((16,), f32)
        return c

    lax.fori_loop(0, _NPW, czero, 0)

    def wchunk(ch, nvec0):
        pltpu.sync_copy(dst_hbm.at[pl.ds(ch * _DCH, _DCH)], dstv)

        def wcomp(i, nvec):
            v = dstv[pl.ds(i * 16, 16)]
            inb = (v >= lo) & (v < lo + _NPW)
            mi = inb.astype(i32)
            p = _prefix16(mi, lane)
            pos = nvec + p - mi
            posw = jnp.minimum(jnp.where(inb, pos, _TLCAP - 1), _TLCAP - 1)
            plsc.store_scatter(tle, [posw], lane + ch * _DCH + i * 16)
            plsc.store_scatter(tld, [posw], v)
            return nvec + _lperm(p, fifteen)

        return lax.fori_loop(0, _DCH // 16, wcomp, nvec0)

    nvec = lax.fori_loop(0, N_EDGES // _DCH, wchunk, jnp.zeros((16,), i32))
    n_t = jnp.minimum(nvec[0], _TLCAP - 1)

    def bucket(k, carry):
        blo = lo + k * _BKN

        def initrow(r, c):
            for j in range(PF // 16):
                sl = pl.ds(j * 16, 16)
                acc_s[r, sl] = jnp.zeros((16,), f32)
                acc_q[r, sl] = jnp.zeros((16,), f32)
                acc_n[r, sl] = jnp.full((16,), _FINF, f32)
                acc_x[r, sl] = jnp.full((16,), -_FINF, f32)
            return c

        lax.fori_loop(0, _BKN, initrow, 0)

        def ezero(i, c):
            eb[pl.ds(i * 16, 16)] = jnp.zeros((16,), i32)
            return c

        lax.fori_loop(0, _BKCAP // 16, ezero, 0)

        def bcomp(i, nbv):
            v = tld[pl.ds(i * 16, 16)]
            e = tle[pl.ds(i * 16, 16)]
            valid = (lane + i * 16) < n_t
            msk = valid & (v >= blo) & (v < blo + _BKN)
            mi = msk.astype(i32)
            p = _prefix16(mi, lane)
            pos = nbv + p - mi
            posw = jnp.minimum(jnp.where(msk, pos, _BKCAP - 1), _BKCAP - 1)
            plsc.store_scatter(eb, [posw], e)
            plsc.store_scatter(lb, [posw], v - blo)
            return nbv + _lperm(p, fifteen)

        nbv = lax.fori_loop(0, (n_t + 15) // 16, bcomp, jnp.zeros((16,), i32))
        n_b = jnp.minimum(nbv[0], _BKCAP - 16)

        def ebatch(bi, c):
            pltpu.async_copy(m4_hbm.at[eb.at[pl.ds(bi * _RB, _RB)]], rows, sem).wait()

            def erow(r, c2):
                pidx = jnp.full((16,), bi * _RB + r, i32)
                l = plsc.load_gather(lb, [pidx])[0]
                ci = k * _BKN + l
                cnt2[ci, :] = cnt2[ci, :] + lane0f
                for j in range(PF // 16):
                    sl = pl.ds(j * 16, 16)
                    mv = rows[r, sl]
                    plsc.addupdate(acc_s.at[l, sl], mv)
                    plsc.addupdate(acc_q.at[l, sl], mv * mv)
                    acc_n[l, sl] = jnp.minimum(acc_n[l, sl], mv)
                    acc_x[l, sl] = jnp.maximum(acc_x[l, sl], mv)
                return c2

            lax.fori_loop(0, jnp.minimum(_RB, n_b - bi * _RB), erow, 0)
            return c

        lax.fori_loop(0, (n_b + _RB - 1) // _RB, ebatch, 0)

        pltpu.sync_copy(acc_s, s_hbm.at[pl.ds(blo, _BKN)])
        pltpu.sync_copy(acc_q, q_hbm.at[pl.ds(blo, _BKN)])
        pltpu.sync_copy(acc_n, n_hbm.at[pl.ds(blo, _BKN)])
        pltpu.sync_copy(acc_x, x_hbm.at[pl.ds(blo, _BKN)])
        return carry

    lax.fori_loop(0, _NBK, bucket, 0)

    def cgath(g, c):
        ridx = lane + g * 16
        zidx = jnp.zeros((16,), i32)
        cntv[pl.ds(g * 16, 16)] = plsc.load_gather(cnt2, [ridx, zidx])
        return c

    lax.fori_loop(0, _NPW // 16, cgath, 0)
    pltpu.sync_copy(cntv, c_hbm.at[pl.ds(lo, _NPW)])


def _aggregate(m4, dst):
    mesh = plsc.VectorSubcoreMesh(core_axis_name="c", subcore_axis_name="s")
    f = functools.partial(
        pl.kernel,
        out_type=[jax.ShapeDtypeStruct((N_NODES, PF), jnp.float32)] * 4
        + [jax.ShapeDtypeStruct((N_NODES,), jnp.float32)],
        mesh=mesh,
        compiler_params=pltpu.CompilerParams(needs_layout_passes=False),
        scratch_types=[
            pltpu.VMEM((_DCH,), jnp.int32),
            pltpu.VMEM((_TLCAP,), jnp.int32),
            pltpu.VMEM((_TLCAP,), jnp.int32),
            pltpu.VMEM((_BKCAP,), jnp.int32),
            pltpu.VMEM((_BKCAP + 16,), jnp.int32),
            pltpu.VMEM((_BKN, PF), jnp.float32),
            pltpu.VMEM((_BKN, PF), jnp.float32),
            pltpu.VMEM((_BKN, PF), jnp.float32),
            pltpu.VMEM((_BKN, PF), jnp.float32),
            pltpu.VMEM((_RB, PF), jnp.float32),
            pltpu.VMEM((_NPW,), jnp.float32),
            pltpu.VMEM((_NPW, 16), jnp.float32),
            pltpu.SemaphoreType.DMA,
        ],
    )(_agg_sc_body)
    s, sq, mn, mx, cnt = f(m4, dst)
    return s, sq, mn, mx, cnt


def kernel(x, edge_index, edge_attr, batch, params):
    f32 = jnp.float32
    xp = _pad2(x, N_NODES, PF).astype(jnp.bfloat16)
    p = params
    w1 = _pad2(p["mlp1"]["w"], PF, PF)
    b1 = _padb(p["mlp1"]["b"], PF)
    pre0w = p["pre"][0]["w"]
    wd = _pad2(pre0w[:F], PF, PF)
    bd = _padb(p["pre"][0]["b"], PF)
    ws_ = _pad2(pre0w[F:2 * F], PF, PF)
    emb = _pad2(p["edge_emb"], 32, 128)
    wenc = _pad2(p["edge_enc"]["w"], 128, PF)
    benc = _padb(p["edge_enc"]["b"], PF)
    wc = _pad2(pre0w[2 * F:], PF, PF)
    prew = [_pad2(p["pre"][i]["w"], PF, PF) for i in range(1, 5)]
    preb = [_padb(p["pre"][i]["b"], PF) for i in range(1, 5)]
    post0w = p["post"][0]["w"]
    wslices = [_pad2(post0w[i * F:(i + 1) * F], PF, PF) for i in range(6)]
    b0 = _padb(p["post"][0]["b"], PF)
    postw = [_pad2(p["post"][i]["w"], PF, PF) for i in range(1, 5)]
    postb = [_padb(p["post"][i]["b"], PF) for i in range(1, 5)]
    wl = _pad2(p["lin"]["w"], PF, PF)
    bl = _padb(p["lin"]["b"], PF)
    gamma = _padb(p["bn_gamma"], PF)
    beta = _padb(p["bn_beta"], PF)
    m2w1 = _pad2(p["mlp2"][0]["w"], PF, 640)
    m2b1 = _padb(p["mlp2"][0]["b"], 640)
    m2w2 = _pad2(p["mlp2"][1]["w"], 640, 128)
    m2b2 = _padb(p["mlp2"][1]["b"], 128)
    m2w3 = _pad2(p["mlp2"][2]["w"], 128, 128)
    m2b3 = _padb(p["mlp2"][2]["b"], 128)
    m3w1 = _pad2(p["mlp3"][0]["w"], PF, 640)
    m3b1 = _padb(p["mlp3"][0]["b"], 640)
    m3w2 = _pad2(p["mlp3"][1]["w"], 640, 128)
    m3b2 = _padb(p["mlp3"][1]["b"], 128)
    m3w3 = _pad2(p["mlp3"][2]["w"], 128, 128)
    m3b3 = _padb(p["mlp3"][2]["b"], 128)
    aw5 = jax.nn.softmax(p["agg_w"])
    aw = jnp.zeros((8, 128), f32).at[0, :5].set(aw5)

    dst = edge_index[1]
    src = edge_index[0]
    attr3 = edge_attr.astype(jnp.int32).reshape(N_EDGES // 768, 1, 768)
    batch3 = batch.astype(jnp.int32).reshape(N_NODES // 512, 1, 512)

    bf = jnp.bfloat16
    w1 = w1.astype(bf)
    wd = wd.astype(bf)
    ws_ = ws_.astype(bf)
    prew = [w.astype(bf) for w in prew]
    wslices = [w.astype(bf) for w in wslices]
    postw = [w.astype(bf) for w in postw]
    wl = wl.astype(bf)
    m3w1 = m3w1.astype(bf)
    m3w2 = m3w2.astype(bf)
    m3w3 = m3w3.astype(bf)
    ctab = _ctab(emb, wenc, benc, wc).astype(bf)
    h0, hd, hs = _node_pre(xp, w1, b1, wd, bd, ws_)
    m0 = _edge_gather(hd, hs, dst, src)
    m4 = _pre_chain(m0, attr3, ctab, prew, preb)
    s, sq, mn, mx, cnt = _aggregate(m4, dst)
    cnt128 = jnp.broadcast_to(cnt[:, None], (N_NODES, 128))
    o0 = _post0(h0, s, sq, mn, mx, cnt128, aw, wslices, b0)
    out, part = _post_chain(o0, postw, postb, wl, bl)
    xf_pad, pool = _finale(out, part, batch3, gamma, beta, m3w1, m3b1, m3w2, m3b2, m3w3, m3b3)
    xe_pad = _mlp2(pool, m2w1, m2b1, m2w2, m2b2, m2w3, m2b3)
    return xf_pad[:, :3], xe_pad[:, :1]
